# baseline (speedup 1.0000x reference)
import functools

import jax
import jax.numpy as jnp
from jax.experimental import pallas as pl

H = 256
ALPHA = 0.02


def _mm_body(x_ref, w_ref, b_ref, o_ref, *, act):
    acc = jnp.dot(x_ref[...], w_ref[...], preferred_element_type=jnp.float32)
    acc = acc + b_ref[...]
    if act == "relu":
        acc = jnp.maximum(acc, 0.0)
    elif act == "leaky":
        acc = jnp.where(acc >= 0.0, acc, ALPHA * acc)
    elif act == "sigmoid":
        acc = jax.nn.sigmoid(acc)
    o_ref[...] = acc


@functools.partial(jax.jit, static_argnames=("act", "bn"))
def _mm(x, wt, b, act="none", bn=1024):
    n, k = x.shape
    d = wt.shape[1]
    bn = min(bn, n)
    grid = (pl.cdiv(n, bn),)
    return pl.pallas_call(
        functools.partial(_mm_body, act=act),
        grid=grid,
        in_specs=[
            pl.BlockSpec((bn, k), lambda i: (i, 0)),
            pl.BlockSpec((k, d), lambda i: (0, 0)),
            pl.BlockSpec((1, d), lambda i: (0, 0)),
        ],
        out_specs=pl.BlockSpec((bn, d), lambda i: (i, 0)),
        out_shape=jax.ShapeDtypeStruct((n, d), jnp.float32),
    )(x, wt, b.reshape(1, d))


def _seg_softmax(logits, seg, num_segments):
    m = jax.ops.segment_max(logits, seg, num_segments)
    m = jnp.where(jnp.isfinite(m), m, 0.0)
    ex = jnp.exp(logits - m[seg])
    s = jax.ops.segment_sum(ex, seg, num_segments)
    return ex / (s[seg] + 1e-16)


def _conv_rel(p, x_src, x_dst, src, dst, ea, n_dst, update_relu):
    w_att1 = p["att1"]["W"]
    w1a, w1b = w_att1[:, :H], w_att1[:, H:]
    b1 = p["att1"]["b"]
    wq, bq = p["Wq"]["W"], p["Wq"]["b"]
    wr, br = p["Wr"]["W"], p["Wr"]["b"]
    wa = jnp.dot(w1a, wq).T
    wb = jnp.dot(w1b, wr).T
    ba = jnp.dot(w1a, bq) + b1
    bb = jnp.dot(w1b, br)
    a_node = _mm(x_dst, wa, ba)
    b_node = _mm(x_src, wb, bb)

    w_msg = p["msg"]["W"]
    m_node = _mm(x_dst, w_msg[:, :H].T, p["msg"]["b"])
    wm_e = w_msg[:, H]

    w2 = p["att2"]["W"][0]
    b2 = p["att2"]["b"][0]

    g = a_node[dst] + b_node[src]
    g = jnp.where(g >= 0.0, g, ALPHA * g)
    logits = g @ w2 + b2
    a = _seg_softmax(logits, dst, n_dst)
    out = jnp.maximum(m_node[dst] + ea * wm_e[None, :], 0.0)
    aggr = jax.ops.segment_sum(a[:, None] * out, dst, n_dst)

    w_upd = p["upd"]["W"]
    upd = _mm(jnp.concatenate([aggr, x_dst], axis=1), w_upd.T, p["upd"]["b"])
    return jnp.maximum(upd, 0.0) if update_relu else upd


def kernel(x_user, x_server, edge_index_s2u, edge_index_u2u, edge_index_u2s,
           edge_attr_s2u, edge_attr_u2u, params):
    n_user = x_user.shape[0]
    s2u_src = edge_index_s2u[0].astype(jnp.int32)
    s2u_dst = edge_index_s2u[1].astype(jnp.int32)
    u2u_src = edge_index_u2u[0].astype(jnp.int32)
    u2u_dst = edge_index_u2u[1].astype(jnp.int32)
    u2s_u = edge_index_u2s[0].astype(jnp.int32)
    u2s_s = edge_index_u2s[1].astype(jnp.int32)

    ue, se = params["user_enc"], params["server_enc"]
    xu = _mm(x_user, ue[0]["W"].T, ue[0]["b"], act="relu")
    xu = _mm(xu, ue[1]["W"].T, ue[1]["b"], act="leaky")
    xs = _mm(x_server, se[0]["W"].T, se[0]["b"], act="relu")
    xs = _mm(xs, se[1]["W"].T, se[1]["b"], act="leaky")

    for cp in params["convs"]:
        u1 = _conv_rel(cp["s2u"], xs, xu, s2u_src, s2u_dst, edge_attr_s2u,
                       n_user, True)
        u2 = _conv_rel(cp["u2u"], xu, xu, u2u_src, u2u_dst, edge_attr_u2u,
                       n_user, False)
        xu = u1 + u2

    off = params["off"]
    o1 = _mm(xu, off[0]["W"].T, off[0]["b"], act="sigmoid")
    ol = _mm(o1, off[1]["W"].T, off[1]["b"])
    probs = jax.nn.softmax(ol, axis=-1)

    sel = params["sel"]
    w0 = sel[0]["W"]
    p_user = (_mm(xu, w0[:, :H].T, sel[0]["b"])
              + probs @ w0[:, H:H + 2].T)
    q_srv = _mm(xs, w0[:, H + 2:].T, jnp.zeros((H,), jnp.float32))

    h1 = jax.nn.sigmoid(p_user[u2s_u] + q_srv[u2s_s])
    h2 = _mm(h1, sel[1]["W"].T, sel[1]["b"], act="sigmoid")
    aw = _mm(h2, sel[2]["W"].T, sel[2]["b"])[:, 0]
    scheme = _seg_softmax(aw, u2s_u, n_user)
    return probs, scheme


# speedup vs baseline: 1.2893x; 1.2893x over previous
import functools

import jax
import jax.numpy as jnp
from jax import lax
from jax.experimental import pallas as pl
from jax.experimental.pallas import tpu as pltpu
from jax.experimental.pallas import tpu_sc as plsc

H = 256
ALPHA = 0.02

NC, NS, L = 2, 16, 16
NW = NC * NS
E = 160000
EW = 5120
E_PAD = EW * NW
C = 128
N_PAD = 10240
PAD_DST = N_PAD - 1


def _vperm(x, idx):
    return lax.gather(
        x, idx[:, None],
        lax.GatherDimensionNumbers(offset_dims=(), collapsed_slice_dims=(0,),
                                   start_index_map=(0,)),
        slice_sizes=(1,), mode=lax.GatherScatterMode.PROMISE_IN_BOUNDS)


def _sc_logits_body(a_hbm, b_hbm, dstg_hbm, srcg_hbm, dsts_hbm, w2_hbm,
                    l_hbm, mx_hbm,
                    dstg_v, srcg_v, dsts_v, w2_v,
                    arows, brows, lbuf, maxacc, sema, semb):
    wid = lax.axis_index("s") * NC + lax.axis_index("c")
    base = wid * EW
    pltpu.sync_copy(dstg_hbm.at[pl.ds(base, EW)], dstg_v)
    pltpu.sync_copy(srcg_hbm.at[pl.ds(base, EW)], srcg_v)
    pltpu.sync_copy(dsts_hbm.at[pl.ds(base, EW)], dsts_v)
    pltpu.sync_copy(w2_hbm, w2_v)
    w2c = [w2_v[pl.ds(16 * i, 16)] for i in range(16)]
    lane = lax.broadcasted_iota(jnp.int32, (L,), 0)
    rots = [(lane + r) & (L - 1) for r in (8, 4, 2, 1)]

    def neg_init(i, _):
        maxacc[pl.ds(i * L, L)] = jnp.full((L,), -1e30, jnp.float32)
        return 0
    lax.fori_loop(0, N_PAD // L, neg_init, 0)

    def chunk_body(c, _):
        cpa = pltpu.async_copy(a_hbm.at[dstg_v.at[pl.ds(c * C, C)]], arows,
                               sema)
        cpb = pltpu.async_copy(b_hbm.at[srcg_v.at[pl.ds(c * C, C)]], brows,
                               semb)
        cpa.wait()
        cpb.wait()

        def edge_body(e, lvec):
            acc = jnp.zeros((L,), jnp.float32)
            for kc in range(16):
                va = arows[e, pl.ds(kc * 16, 16)]
                vb = brows[e, pl.ds(kc * 16, 16)]
                g = va + vb
                g = jnp.where(g >= 0.0, g, ALPHA * g)
                acc = acc + g * w2c[kc]
            for r in rots:
                acc = acc + _vperm(acc, r)
            lvec = lvec + jnp.where(lane == (e % L), acc, 0.0)

            @pl.when(e % L == L - 1)
            def _():
                lbuf[pl.ds(c * C + (e // L) * L, L)] = lvec
            return jnp.where(e % L == L - 1, jnp.zeros((L,), jnp.float32),
                             lvec)
        lax.fori_loop(0, C, edge_body, jnp.zeros((L,), jnp.float32))
        return 0
    lax.fori_loop(0, EW // C, chunk_body, 0)

    def seg_max(gi, _):
        lvec = lbuf[pl.ds(gi * L, L)]
        dstv = dsts_v[pl.ds(gi * L, L)]
        k, v = dstv, lvec
        for r in range(1, L):
            ridx = (lane + r) & (L - 1)
            km = _vperm(k, ridx)
            vm = _vperm(v, ridx)
            v = jnp.where(km == k, jnp.maximum(v, vm), v)
        cur = plsc.load_gather(maxacc, [k])
        plsc.store_scatter(maxacc, [k], jnp.maximum(cur, v))
        return 0
    lax.fori_loop(0, EW // L, seg_max, 0)

    pltpu.sync_copy(lbuf, l_hbm.at[pl.ds(base, EW)])
    pltpu.sync_copy(maxacc, mx_hbm.at[pl.ds(wid * N_PAD, N_PAD)])


def _sc_logits(a_tab, b_tab, dstg, srcg, dsts, w2):
    return pl.kernel(
        _sc_logits_body,
        out_type=(jax.ShapeDtypeStruct((E_PAD,), jnp.float32),
                  jax.ShapeDtypeStruct((NW * N_PAD,), jnp.float32)),
        mesh=plsc.VectorSubcoreMesh(core_axis_name="c", subcore_axis_name="s"),
        compiler_params=pltpu.CompilerParams(needs_layout_passes=False),
        scratch_types=[
            pltpu.VMEM((EW,), jnp.int32),
            pltpu.VMEM((EW,), jnp.int32),
            pltpu.VMEM((EW,), jnp.int32),
            pltpu.VMEM((H,), jnp.float32),
            pltpu.VMEM((C, H), jnp.float32),
            pltpu.VMEM((C, H), jnp.float32),
            pltpu.VMEM((EW,), jnp.float32),
            pltpu.VMEM((N_PAD,), jnp.float32),
            pltpu.SemaphoreType.DMA,
            pltpu.SemaphoreType.DMA,
        ],
    )(a_tab, b_tab, dstg, srcg, dsts, w2)


def _mm_body(x_ref, w_ref, b_ref, o_ref, *, act):
    acc = jnp.dot(x_ref[...], w_ref[...], preferred_element_type=jnp.float32)
    acc = acc + b_ref[...]
    if act == "relu":
        acc = jnp.maximum(acc, 0.0)
    elif act == "leaky":
        acc = jnp.where(acc >= 0.0, acc, ALPHA * acc)
    elif act == "sigmoid":
        acc = jax.nn.sigmoid(acc)
    o_ref[...] = acc


@functools.partial(jax.jit, static_argnames=("act", "bn"))
def _mm(x, wt, b, act="none", bn=1024):
    n, k = x.shape
    d = wt.shape[1]
    bn = min(bn, n)
    grid = (pl.cdiv(n, bn),)
    return pl.pallas_call(
        functools.partial(_mm_body, act=act),
        grid=grid,
        in_specs=[
            pl.BlockSpec((bn, k), lambda i: (i, 0)),
            pl.BlockSpec((k, d), lambda i: (0, 0)),
            pl.BlockSpec((1, d), lambda i: (0, 0)),
        ],
        out_specs=pl.BlockSpec((bn, d), lambda i: (i, 0)),
        out_shape=jax.ShapeDtypeStruct((n, d), jnp.float32),
    )(x, wt, b.reshape(1, d))


def _seg_softmax(logits, seg, num_segments):
    m = jax.ops.segment_max(logits, seg, num_segments)
    m = jnp.where(jnp.isfinite(m), m, 0.0)
    ex = jnp.exp(logits - m[seg])
    s = jax.ops.segment_sum(ex, seg, num_segments)
    return ex / (s[seg] + 1e-16)


def _pad_edges(src, dst):
    pad = E_PAD - src.shape[0]
    src_g = jnp.concatenate([src, jnp.zeros((pad,), jnp.int32)])
    dst_g = jnp.concatenate([dst, jnp.zeros((pad,), jnp.int32)])
    dst_s = jnp.concatenate([dst, jnp.full((pad,), PAD_DST, jnp.int32)])
    return src_g, dst_g, dst_s


def _conv_rel(p, x_src, x_dst, src, dst, ea, n_dst, update_relu):
    w_att1 = p["att1"]["W"]
    w1a, w1b = w_att1[:, :H], w_att1[:, H:]
    b1 = p["att1"]["b"]
    wq, bq = p["Wq"]["W"], p["Wq"]["b"]
    wr, br = p["Wr"]["W"], p["Wr"]["b"]
    wa = jnp.dot(w1a, wq).T
    wb = jnp.dot(w1b, wr).T
    ba = jnp.dot(w1a, bq) + b1
    bb = jnp.dot(w1b, br)
    a_node = _mm(x_dst, wa, ba)
    b_node = _mm(x_src, wb, bb)

    w_msg = p["msg"]["W"]
    m_node = _mm(x_dst, w_msg[:, :H].T, p["msg"]["b"])
    wm_e = w_msg[:, H]

    w2 = p["att2"]["W"][0]

    src_g, dst_g, dst_s = _pad_edges(src, dst)
    l_pad, maxes = _sc_logits(a_node, b_node, dst_g, src_g, dst_s, w2)
    logits = l_pad[:E]
    m = maxes.reshape(NW, N_PAD).max(axis=0)
    ex = jnp.exp(logits - m[dst])
    s = jax.ops.segment_sum(ex, dst, n_dst)
    a = ex / (s[dst] + 1e-16)
    out = jnp.maximum(m_node[dst] + ea * wm_e[None, :], 0.0)
    aggr = jax.ops.segment_sum(a[:, None] * out, dst, n_dst)

    w_upd = p["upd"]["W"]
    upd = _mm(jnp.concatenate([aggr, x_dst], axis=1), w_upd.T, p["upd"]["b"])
    return jnp.maximum(upd, 0.0) if update_relu else upd


def kernel(x_user, x_server, edge_index_s2u, edge_index_u2u, edge_index_u2s,
           edge_attr_s2u, edge_attr_u2u, params):
    n_user = x_user.shape[0]
    s2u_src = edge_index_s2u[0].astype(jnp.int32)
    s2u_dst = edge_index_s2u[1].astype(jnp.int32)
    u2u_src = edge_index_u2u[0].astype(jnp.int32)
    u2u_dst = edge_index_u2u[1].astype(jnp.int32)
    u2s_u = edge_index_u2s[0].astype(jnp.int32)
    u2s_s = edge_index_u2s[1].astype(jnp.int32)

    ue, se = params["user_enc"], params["server_enc"]
    xu = _mm(x_user, ue[0]["W"].T, ue[0]["b"], act="relu")
    xu = _mm(xu, ue[1]["W"].T, ue[1]["b"], act="leaky")
    xs = _mm(x_server, se[0]["W"].T, se[0]["b"], act="relu")
    xs = _mm(xs, se[1]["W"].T, se[1]["b"], act="leaky")

    for cp in params["convs"]:
        u1 = _conv_rel(cp["s2u"], xs, xu, s2u_src, s2u_dst, edge_attr_s2u,
                       n_user, True)
        u2 = _conv_rel(cp["u2u"], xu, xu, u2u_src, u2u_dst, edge_attr_u2u,
                       n_user, False)
        xu = u1 + u2

    off = params["off"]
    o1 = _mm(xu, off[0]["W"].T, off[0]["b"], act="sigmoid")
    ol = _mm(o1, off[1]["W"].T, off[1]["b"])
    probs = jax.nn.softmax(ol, axis=-1)

    sel = params["sel"]
    w0 = sel[0]["W"]
    p_user = (_mm(xu, w0[:, :H].T, sel[0]["b"])
              + probs @ w0[:, H:H + 2].T)
    q_srv = _mm(xs, w0[:, H + 2:].T, jnp.zeros((H,), jnp.float32))

    h1 = jax.nn.sigmoid(p_user[u2s_u] + q_srv[u2s_s])
    h2 = _mm(h1, sel[1]["W"].T, sel[1]["b"], act="sigmoid")
    aw = _mm(h2, sel[2]["W"].T, sel[2]["b"])[:, 0]
    scheme = _seg_softmax(aw, u2s_u, n_user)
    return probs, scheme


# speedup vs baseline: 1.7103x; 1.3265x over previous
import functools

import jax
import jax.numpy as jnp
from jax import lax
from jax.experimental import pallas as pl
from jax.experimental.pallas import tpu as pltpu
from jax.experimental.pallas import tpu_sc as plsc

H = 256
ALPHA = 0.02

NC, NS, L = 2, 16, 16
NW = NC * NS
E = 160000
EW = 5120
E_PAD = EW * NW
C = 128
N_PAD = 10240
PAD_DST = N_PAD - 1
QH = 64


def _vperm(x, idx):
    return lax.gather(
        x, idx[:, None],
        lax.GatherDimensionNumbers(offset_dims=(), collapsed_slice_dims=(0,),
                                   start_index_map=(0,)),
        slice_sizes=(1,), mode=lax.GatherScatterMode.PROMISE_IN_BOUNDS)


def _sc_logits_body(a_hbm, b_hbm, dstg_hbm, srcg_hbm, dsts_hbm, w2_hbm,
                    l_hbm, mx_hbm,
                    dstg_v, srcg_v, dsts_v, w2_v,
                    arows, brows, lbuf, maxacc, sema, semb):
    wid = lax.axis_index("s") * NC + lax.axis_index("c")
    base = wid * EW
    pltpu.sync_copy(dstg_hbm.at[pl.ds(base, EW)], dstg_v)
    pltpu.sync_copy(srcg_hbm.at[pl.ds(base, EW)], srcg_v)
    pltpu.sync_copy(dsts_hbm.at[pl.ds(base, EW)], dsts_v)
    pltpu.sync_copy(w2_hbm, w2_v)
    w2c = [w2_v[pl.ds(16 * i, 16)] for i in range(16)]
    lane = lax.broadcasted_iota(jnp.int32, (L,), 0)
    rots = [(lane + r) & (L - 1) for r in (8, 4, 2, 1)]

    def neg_init(i, _):
        maxacc[pl.ds(i * L, L)] = jnp.full((L,), -1e30, jnp.float32)
        return 0
    lax.fori_loop(0, N_PAD // L, neg_init, 0)

    def chunk_body(c, _):
        cpa = pltpu.async_copy(a_hbm.at[dstg_v.at[pl.ds(c * C, C)]], arows,
                               sema)
        cpb = pltpu.async_copy(b_hbm.at[srcg_v.at[pl.ds(c * C, C)]], brows,
                               semb)
        cpa.wait()
        cpb.wait()

        def edge_body(e, lvec):
            acc = jnp.zeros((L,), jnp.float32)
            for kc in range(16):
                va = arows[e, pl.ds(kc * 16, 16)]
                vb = brows[e, pl.ds(kc * 16, 16)]
                g = va + vb
                g = jnp.where(g >= 0.0, g, ALPHA * g)
                acc = acc + g * w2c[kc]
            for r in rots:
                acc = acc + _vperm(acc, r)
            lvec = lvec + jnp.where(lane == (e % L), acc, 0.0)

            @pl.when(e % L == L - 1)
            def _():
                lbuf[pl.ds(c * C + (e // L) * L, L)] = lvec
            return jnp.where(e % L == L - 1, jnp.zeros((L,), jnp.float32),
                             lvec)
        lax.fori_loop(0, C, edge_body, jnp.zeros((L,), jnp.float32))
        return 0
    lax.fori_loop(0, EW // C, chunk_body, 0)

    def seg_max(gi, _):
        lvec = lbuf[pl.ds(gi * L, L)]
        dstv = dsts_v[pl.ds(gi * L, L)]
        k, v = dstv, lvec
        for r in range(1, L):
            ridx = (lane + r) & (L - 1)
            km = _vperm(k, ridx)
            vm = _vperm(v, ridx)
            v = jnp.where(km == k, jnp.maximum(v, vm), v)
        cur = plsc.load_gather(maxacc, [k])
        plsc.store_scatter(maxacc, [k], jnp.maximum(cur, v))
        return 0
    lax.fori_loop(0, EW // L, seg_max, 0)

    pltpu.sync_copy(lbuf, l_hbm.at[pl.ds(base, EW)])
    pltpu.sync_copy(maxacc, mx_hbm.at[pl.ds(wid * N_PAD, N_PAD)])


def _sc_logits(a_tab, b_tab, dstg, srcg, dsts, w2):
    return pl.kernel(
        _sc_logits_body,
        out_type=(jax.ShapeDtypeStruct((E_PAD,), jnp.float32),
                  jax.ShapeDtypeStruct((NW * N_PAD,), jnp.float32)),
        mesh=plsc.VectorSubcoreMesh(core_axis_name="c", subcore_axis_name="s"),
        compiler_params=pltpu.CompilerParams(needs_layout_passes=False),
        scratch_types=[
            pltpu.VMEM((EW,), jnp.int32),
            pltpu.VMEM((EW,), jnp.int32),
            pltpu.VMEM((EW,), jnp.int32),
            pltpu.VMEM((H,), jnp.float32),
            pltpu.VMEM((C, H), jnp.float32),
            pltpu.VMEM((C, H), jnp.float32),
            pltpu.VMEM((EW,), jnp.float32),
            pltpu.VMEM((N_PAD,), jnp.float32),
            pltpu.SemaphoreType.DMA,
            pltpu.SemaphoreType.DMA,
        ],
    )(a_tab, b_tab, dstg, srcg, dsts, w2)


def _mm_body(x_ref, w_ref, b_ref, o_ref, *, act):
    acc = jnp.dot(x_ref[...], w_ref[...], preferred_element_type=jnp.float32)
    acc = acc + b_ref[...]
    if act == "relu":
        acc = jnp.maximum(acc, 0.0)
    elif act == "leaky":
        acc = jnp.where(acc >= 0.0, acc, ALPHA * acc)
    elif act == "sigmoid":
        acc = jax.nn.sigmoid(acc)
    o_ref[...] = acc


@functools.partial(jax.jit, static_argnames=("act", "bn"))
def _mm(x, wt, b, act="none", bn=1024):
    n, k = x.shape
    d = wt.shape[1]
    bn = min(bn, n)
    grid = (pl.cdiv(n, bn),)
    return pl.pallas_call(
        functools.partial(_mm_body, act=act),
        grid=grid,
        in_specs=[
            pl.BlockSpec((bn, k), lambda i: (i, 0)),
            pl.BlockSpec((k, d), lambda i: (0, 0)),
            pl.BlockSpec((1, d), lambda i: (0, 0)),
        ],
        out_specs=pl.BlockSpec((bn, d), lambda i: (i, 0)),
        out_shape=jax.ShapeDtypeStruct((n, d), jnp.float32),
    )(x, wt, b.reshape(1, d))


def _seg_softmax(logits, seg, num_segments):
    m = jax.ops.segment_max(logits, seg, num_segments)
    m = jnp.where(jnp.isfinite(m), m, 0.0)
    ex = jnp.exp(logits - m[seg])
    s = jax.ops.segment_sum(ex, seg, num_segments)
    return ex / (s[seg] + 1e-16)


def _sc_aggr_body(l_hbm, mx_hbm, dsts_hbm, dstg_hbm, ea_hbm, m0_hbm, m1_hbm,
                  wm_hbm, zeros_hbm,
                  s_hbm, ag_hbm,
                  m_buf, s_acc, mx_v, m_rows, idxw_v, idxg_v, lc_v, eac_v,
                  wm_v, acc_sp, sema):
    cid = lax.axis_index("c")
    sid = lax.axis_index("s")
    wid = sid * NC + cid
    base = wid * EW

    pltpu.sync_copy(wm_hbm, wm_v)

    def mx_blk(blk, _):
        pltpu.sync_copy(mx_hbm.at[:, pl.ds(blk * 128, 128)], mx_v)

        def mx_grp(i, _):
            m = mx_v[0, pl.ds(i * L, L)]
            for w in range(1, NW):
                m = jnp.maximum(m, mx_v[w, pl.ds(i * L, L)])
            m_buf[pl.ds(blk * 128 + i * L, L)] = m
            return 0
        lax.fori_loop(0, 128 // L, mx_grp, 0)
        return 0
    lax.fori_loop(0, N_PAD // 128, mx_blk, 0)

    def z_init(i, _):
        s_acc[pl.ds(i * L, L)] = jnp.zeros((L,), jnp.float32)
        return 0
    lax.fori_loop(0, N_PAD // L, z_init, 0)

    for h in range(2):
        m_tab = m0_hbm if h == 0 else m1_hbm
        pltpu.sync_copy(zeros_hbm, acc_sp.at[pl.ds(sid * 640, 640)])
        plsc.subcore_barrier()

        def chunk_body(c, _):
            pltpu.sync_copy(dsts_hbm.at[pl.ds(base + c * C, C)], idxw_v)
            pltpu.sync_copy(dstg_hbm.at[pl.ds(base + c * C, C)], idxg_v)
            pltpu.sync_copy(l_hbm.at[pl.ds(base + c * C, C)], lc_v)
            pltpu.sync_copy(ea_hbm.at[pl.ds(base + c * C, C)], eac_v)
            pltpu.async_copy(m_tab.at[idxg_v], m_rows, sema).wait()

            def grp_body(g, _):
                dv = idxw_v[pl.ds(g * L, L)]
                lv = lc_v[pl.ds(g * L, L)]
                mv = plsc.load_gather(m_buf, [dv])
                ex = jnp.exp(lv - mv)
                if h == 0:
                    plsc.addupdate_scatter(s_acc, [dv], ex)
                eag = eac_v[pl.ds(g * L, L)]
                for el in range(L):
                    e = g * L + el
                    sel = jnp.full((L,), el, jnp.int32)
                    eav = _vperm(eag, sel)
                    exv = _vperm(ex, sel)
                    for kc in range(128 // L):
                        mr = m_rows[e, pl.ds(kc * L, L)]
                        wmv = wm_v[pl.ds(h * 128 + kc * L, L)]
                        m_rows[e, pl.ds(kc * L, L)] = (
                            jnp.maximum(mr + eav * wmv, 0.0) * exv)
                return 0
            lax.fori_loop(0, C // L, grp_body, 0)
            pltpu.sync_copy(m_rows, acc_sp.at[idxw_v], add=True)
            return 0
        lax.fori_loop(0, EW // C, chunk_body, 0)
        plsc.subcore_barrier()
        pltpu.sync_copy(
            acc_sp.at[pl.ds(sid * 640, 640)],
            ag_hbm.at[pl.ds((h * NC + cid) * N_PAD + sid * 640, 640)])
        plsc.subcore_barrier()
    pltpu.sync_copy(s_acc, s_hbm.at[pl.ds(wid * N_PAD, N_PAD)])


def _sc_aggr(l_pad, maxes, dsts, dstg, ea_pad, m_tabs, wm, zeros):
    return pl.kernel(
        _sc_aggr_body,
        out_type=(jax.ShapeDtypeStruct((NW * N_PAD,), jnp.float32),
                  jax.ShapeDtypeStruct((2 * NC * N_PAD, 128), jnp.float32)),
        mesh=plsc.VectorSubcoreMesh(core_axis_name="c", subcore_axis_name="s"),
        compiler_params=pltpu.CompilerParams(needs_layout_passes=False),
        scratch_types=[
            pltpu.VMEM((N_PAD,), jnp.float32),
            pltpu.VMEM((N_PAD,), jnp.float32),
            pltpu.VMEM((NW, 128), jnp.float32),
            pltpu.VMEM((C, 128), jnp.float32),
            pltpu.VMEM((C,), jnp.int32),
            pltpu.VMEM((C,), jnp.int32),
            pltpu.VMEM((C,), jnp.float32),
            pltpu.VMEM((C,), jnp.float32),
            pltpu.VMEM((H,), jnp.float32),
            pltpu.VMEM_SHARED((N_PAD, 128), jnp.float32),
            pltpu.SemaphoreType.DMA,
        ],
    )(l_pad, maxes, dsts, dstg, ea_pad, *m_tabs, wm, zeros)


def _pad_edges(src, dst):
    pad = E_PAD - src.shape[0]
    src_g = jnp.concatenate([src, jnp.zeros((pad,), jnp.int32)])
    dst_g = jnp.concatenate([dst, jnp.zeros((pad,), jnp.int32)])
    dst_s = jnp.concatenate([dst, jnp.full((pad,), PAD_DST, jnp.int32)])
    return src_g, dst_g, dst_s


def _conv_rel(p, x_src, x_dst, src, dst, ea, n_dst, update_relu):
    w_att1 = p["att1"]["W"]
    w1a, w1b = w_att1[:, :H], w_att1[:, H:]
    b1 = p["att1"]["b"]
    wq, bq = p["Wq"]["W"], p["Wq"]["b"]
    wr, br = p["Wr"]["W"], p["Wr"]["b"]
    wa = jnp.dot(w1a, wq).T
    wb = jnp.dot(w1b, wr).T
    ba = jnp.dot(w1a, bq) + b1
    bb = jnp.dot(w1b, br)
    a_node = _mm(x_dst, wa, ba)
    b_node = _mm(x_src, wb, bb)

    w_msg = p["msg"]["W"]
    m_node = _mm(x_dst, w_msg[:, :H].T, p["msg"]["b"])
    wm_e = w_msg[:, H]

    w2 = p["att2"]["W"][0]

    src_g, dst_g, dst_s = _pad_edges(src, dst)
    l_pad, maxes = _sc_logits(a_node, b_node, dst_g, src_g, dst_s, w2)

    ea_pad = jnp.concatenate(
        [ea[:, 0], jnp.zeros((E_PAD - E,), jnp.float32)])
    m_tabs = [m_node[:, :128], m_node[:, 128:]]
    zeros = jnp.zeros((640, 128), jnp.float32)
    s_part, ag_part = _sc_aggr(l_pad, maxes.reshape(NW, N_PAD), dst_s, dst_g,
                               ea_pad, m_tabs, wm_e, zeros)
    s = s_part.reshape(NW, N_PAD).sum(axis=0)[:n_dst]
    ap = ag_part.reshape(2, NC, N_PAD, 128)
    aggr_raw = jnp.concatenate([ap[0, 0] + ap[0, 1], ap[1, 0] + ap[1, 1]],
                               axis=1)[:n_dst]
    aggr = aggr_raw / (s + 1e-16)[:, None]

    w_upd = p["upd"]["W"]
    upd = _mm(jnp.concatenate([aggr, x_dst], axis=1), w_upd.T, p["upd"]["b"])
    return jnp.maximum(upd, 0.0) if update_relu else upd


def kernel(x_user, x_server, edge_index_s2u, edge_index_u2u, edge_index_u2s,
           edge_attr_s2u, edge_attr_u2u, params):
    n_user = x_user.shape[0]
    s2u_src = edge_index_s2u[0].astype(jnp.int32)
    s2u_dst = edge_index_s2u[1].astype(jnp.int32)
    u2u_src = edge_index_u2u[0].astype(jnp.int32)
    u2u_dst = edge_index_u2u[1].astype(jnp.int32)
    u2s_u = edge_index_u2s[0].astype(jnp.int32)
    u2s_s = edge_index_u2s[1].astype(jnp.int32)

    ue, se = params["user_enc"], params["server_enc"]
    xu = _mm(x_user, ue[0]["W"].T, ue[0]["b"], act="relu")
    xu = _mm(xu, ue[1]["W"].T, ue[1]["b"], act="leaky")
    xs = _mm(x_server, se[0]["W"].T, se[0]["b"], act="relu")
    xs = _mm(xs, se[1]["W"].T, se[1]["b"], act="leaky")

    for cp in params["convs"]:
        u1 = _conv_rel(cp["s2u"], xs, xu, s2u_src, s2u_dst, edge_attr_s2u,
                       n_user, True)
        u2 = _conv_rel(cp["u2u"], xu, xu, u2u_src, u2u_dst, edge_attr_u2u,
                       n_user, False)
        xu = u1 + u2

    off = params["off"]
    o1 = _mm(xu, off[0]["W"].T, off[0]["b"], act="sigmoid")
    ol = _mm(o1, off[1]["W"].T, off[1]["b"])
    probs = jax.nn.softmax(ol, axis=-1)

    sel = params["sel"]
    w0 = sel[0]["W"]
    p_user = (_mm(xu, w0[:, :H].T, sel[0]["b"])
              + probs @ w0[:, H:H + 2].T)
    q_srv = _mm(xs, w0[:, H + 2:].T, jnp.zeros((H,), jnp.float32))

    h1 = jax.nn.sigmoid(p_user[u2s_u] + q_srv[u2s_s])
    h2 = _mm(h1, sel[1]["W"].T, sel[1]["b"], act="sigmoid")
    aw = _mm(h2, sel[2]["W"].T, sel[2]["b"])[:, 0]
    scheme = _seg_softmax(aw, u2s_u, n_user)
    return probs, scheme


# speedup vs baseline: 1.9251x; 1.1256x over previous
import functools

import jax
import jax.numpy as jnp
from jax import lax
from jax.experimental import pallas as pl
from jax.experimental.pallas import tpu as pltpu
from jax.experimental.pallas import tpu_sc as plsc

H = 256
ALPHA = 0.02

NC, NS, L = 2, 16, 16
NW = NC * NS
E = 160000
EW = 5120
E_PAD = EW * NW
C = 128
N_PAD = 10240
PAD_DST = N_PAD - 1
QH = 64


def _vperm(x, idx):
    return lax.gather(
        x, idx[:, None],
        lax.GatherDimensionNumbers(offset_dims=(), collapsed_slice_dims=(0,),
                                   start_index_map=(0,)),
        slice_sizes=(1,), mode=lax.GatherScatterMode.PROMISE_IN_BOUNDS)


def _sc_logits_body(a_hbm, b_hbm, dstg_hbm, srcg_hbm, dsts_hbm, w2_hbm,
                    l_hbm, mx_hbm,
                    dstg_v, srcg_v, dsts_v, w2_v,
                    arows, brows, lbuf, maxacc, sema, semb):
    wid = lax.axis_index("s") * NC + lax.axis_index("c")
    base = wid * EW
    pltpu.sync_copy(dstg_hbm.at[pl.ds(base, EW)], dstg_v)
    pltpu.sync_copy(srcg_hbm.at[pl.ds(base, EW)], srcg_v)
    pltpu.sync_copy(dsts_hbm.at[pl.ds(base, EW)], dsts_v)
    pltpu.sync_copy(w2_hbm, w2_v)
    w2c = [w2_v[pl.ds(16 * i, 16)] for i in range(16)]
    lane = lax.broadcasted_iota(jnp.int32, (L,), 0)
    rots = [(lane + r) & (L - 1) for r in (8, 4, 2, 1)]

    def neg_init(i, _):
        maxacc[pl.ds(i * L, L)] = jnp.full((L,), -1e30, jnp.float32)
        return 0
    lax.fori_loop(0, N_PAD // L, neg_init, 0)

    def chunk_body(c, _):
        cpa = pltpu.async_copy(a_hbm.at[dstg_v.at[pl.ds(c * C, C)]], arows,
                               sema)
        cpb = pltpu.async_copy(b_hbm.at[srcg_v.at[pl.ds(c * C, C)]], brows,
                               semb)
        cpa.wait()
        cpb.wait()

        def edge_body(e, lvec):
            acc = jnp.zeros((L,), jnp.float32)
            for kc in range(16):
                va = arows[e, pl.ds(kc * 16, 16)]
                vb = brows[e, pl.ds(kc * 16, 16)]
                g = va + vb
                g = jnp.where(g >= 0.0, g, ALPHA * g)
                acc = acc + g * w2c[kc]
            for r in rots:
                acc = acc + _vperm(acc, r)
            lvec = lvec + jnp.where(lane == (e % L), acc, 0.0)

            @pl.when(e % L == L - 1)
            def _():
                lbuf[pl.ds(c * C + (e // L) * L, L)] = lvec
            return jnp.where(e % L == L - 1, jnp.zeros((L,), jnp.float32),
                             lvec)
        lax.fori_loop(0, C, edge_body, jnp.zeros((L,), jnp.float32))
        return 0
    lax.fori_loop(0, EW // C, chunk_body, 0)

    def seg_max(gi, _):
        lvec = lbuf[pl.ds(gi * L, L)]
        dstv = dsts_v[pl.ds(gi * L, L)]
        k, v = dstv, lvec
        for r in range(1, L):
            ridx = (lane + r) & (L - 1)
            km = _vperm(k, ridx)
            vm = _vperm(v, ridx)
            v = jnp.where(km == k, jnp.maximum(v, vm), v)
        cur = plsc.load_gather(maxacc, [k])
        plsc.store_scatter(maxacc, [k], jnp.maximum(cur, v))
        return 0
    lax.fori_loop(0, EW // L, seg_max, 0)

    pltpu.sync_copy(lbuf, l_hbm.at[pl.ds(base, EW)])
    pltpu.sync_copy(maxacc, mx_hbm.at[pl.ds(wid * N_PAD, N_PAD)])


def _sc_logits(a_tab, b_tab, dstg, srcg, dsts, w2):
    return pl.kernel(
        _sc_logits_body,
        out_type=(jax.ShapeDtypeStruct((E_PAD,), jnp.float32),
                  jax.ShapeDtypeStruct((NW * N_PAD,), jnp.float32)),
        mesh=plsc.VectorSubcoreMesh(core_axis_name="c", subcore_axis_name="s"),
        compiler_params=pltpu.CompilerParams(needs_layout_passes=False),
        scratch_types=[
            pltpu.VMEM((EW,), jnp.int32),
            pltpu.VMEM((EW,), jnp.int32),
            pltpu.VMEM((EW,), jnp.int32),
            pltpu.VMEM((H,), jnp.float32),
            pltpu.VMEM((C, H), jnp.float32),
            pltpu.VMEM((C, H), jnp.float32),
            pltpu.VMEM((EW,), jnp.float32),
            pltpu.VMEM((N_PAD,), jnp.float32),
            pltpu.SemaphoreType.DMA,
            pltpu.SemaphoreType.DMA,
        ],
    )(a_tab, b_tab, dstg, srcg, dsts, w2)


def _mm_body(x_ref, w_ref, b_ref, o_ref, *, act):
    acc = jnp.dot(x_ref[...], w_ref[...], preferred_element_type=jnp.float32)
    acc = acc + b_ref[...]
    if act == "relu":
        acc = jnp.maximum(acc, 0.0)
    elif act == "leaky":
        acc = jnp.where(acc >= 0.0, acc, ALPHA * acc)
    elif act == "sigmoid":
        acc = jax.nn.sigmoid(acc)
    o_ref[...] = acc


@functools.partial(jax.jit, static_argnames=("act", "bn"))
def _mm(x, wt, b, act="none", bn=1024):
    n, k = x.shape
    d = wt.shape[1]
    bn = min(bn, n)
    grid = (pl.cdiv(n, bn),)
    return pl.pallas_call(
        functools.partial(_mm_body, act=act),
        grid=grid,
        in_specs=[
            pl.BlockSpec((bn, k), lambda i: (i, 0)),
            pl.BlockSpec((k, d), lambda i: (0, 0)),
            pl.BlockSpec((1, d), lambda i: (0, 0)),
        ],
        out_specs=pl.BlockSpec((bn, d), lambda i: (i, 0)),
        out_shape=jax.ShapeDtypeStruct((n, d), jnp.float32),
    )(x, wt, b.reshape(1, d))


def _seg_softmax(logits, seg, num_segments):
    m = jax.ops.segment_max(logits, seg, num_segments)
    m = jnp.where(jnp.isfinite(m), m, 0.0)
    ex = jnp.exp(logits - m[seg])
    s = jax.ops.segment_sum(ex, seg, num_segments)
    return ex / (s[seg] + 1e-16)


def _sc_aggr_body(l_hbm, mx_hbm, dsts_hbm, dstg_hbm, ea_hbm, m0_hbm, m1_hbm,
                  wm_hbm, zeros_hbm,
                  s_hbm, ag_hbm,
                  m_buf, s_acc, mx_v, m_rows, idxw_v, idxg_v, lc_v, eac_v,
                  wm_v, acc_sp, sema):
    cid = lax.axis_index("c")
    sid = lax.axis_index("s")
    wid = sid * NC + cid
    base = wid * EW

    pltpu.sync_copy(wm_hbm, wm_v)

    def mx_blk(blk, _):
        pltpu.sync_copy(mx_hbm.at[:, pl.ds(blk * 128, 128)], mx_v)

        def mx_grp(i, _):
            m = mx_v[0, pl.ds(i * L, L)]
            for w in range(1, NW):
                m = jnp.maximum(m, mx_v[w, pl.ds(i * L, L)])
            m_buf[pl.ds(blk * 128 + i * L, L)] = m
            return 0
        lax.fori_loop(0, 128 // L, mx_grp, 0)
        return 0
    lax.fori_loop(0, N_PAD // 128, mx_blk, 0)

    def z_init(i, _):
        s_acc[pl.ds(i * L, L)] = jnp.zeros((L,), jnp.float32)
        return 0
    lax.fori_loop(0, N_PAD // L, z_init, 0)

    for h in range(2):
        m_tab = m0_hbm if h == 0 else m1_hbm
        pltpu.sync_copy(zeros_hbm, acc_sp.at[pl.ds(sid * 640, 640)])
        plsc.subcore_barrier()

        def chunk_body(c, _):
            pltpu.sync_copy(dsts_hbm.at[pl.ds(base + c * C, C)], idxw_v)
            pltpu.sync_copy(dstg_hbm.at[pl.ds(base + c * C, C)], idxg_v)
            pltpu.sync_copy(l_hbm.at[pl.ds(base + c * C, C)], lc_v)
            pltpu.sync_copy(ea_hbm.at[pl.ds(base + c * C, C)], eac_v)
            pltpu.async_copy(m_tab.at[idxg_v], m_rows, sema).wait()

            def grp_body(g, _):
                dv = idxw_v[pl.ds(g * L, L)]
                lv = lc_v[pl.ds(g * L, L)]
                mv = plsc.load_gather(m_buf, [dv])
                ex = jnp.exp(lv - mv)
                if h == 0:
                    plsc.addupdate_scatter(s_acc, [dv], ex)
                eag = eac_v[pl.ds(g * L, L)]
                for el in range(L):
                    e = g * L + el
                    sel = jnp.full((L,), el, jnp.int32)
                    eav = _vperm(eag, sel)
                    exv = _vperm(ex, sel)
                    for kc in range(128 // L):
                        mr = m_rows[e, pl.ds(kc * L, L)]
                        wmv = wm_v[pl.ds(h * 128 + kc * L, L)]
                        m_rows[e, pl.ds(kc * L, L)] = (
                            jnp.maximum(mr + eav * wmv, 0.0) * exv)
                return 0
            lax.fori_loop(0, C // L, grp_body, 0)
            pltpu.sync_copy(m_rows, acc_sp.at[idxw_v], add=True)
            return 0
        lax.fori_loop(0, EW // C, chunk_body, 0)
        plsc.subcore_barrier()
        pltpu.sync_copy(
            acc_sp.at[pl.ds(sid * 640, 640)],
            ag_hbm.at[pl.ds((h * NC + cid) * N_PAD + sid * 640, 640)])
        plsc.subcore_barrier()
    pltpu.sync_copy(s_acc, s_hbm.at[pl.ds(wid * N_PAD, N_PAD)])


def _sc_aggr(l_pad, maxes, dsts, dstg, ea_pad, m_tabs, wm, zeros):
    return pl.kernel(
        _sc_aggr_body,
        out_type=(jax.ShapeDtypeStruct((NW * N_PAD,), jnp.float32),
                  jax.ShapeDtypeStruct((2 * NC * N_PAD, 128), jnp.float32)),
        mesh=plsc.VectorSubcoreMesh(core_axis_name="c", subcore_axis_name="s"),
        compiler_params=pltpu.CompilerParams(needs_layout_passes=False),
        scratch_types=[
            pltpu.VMEM((N_PAD,), jnp.float32),
            pltpu.VMEM((N_PAD,), jnp.float32),
            pltpu.VMEM((NW, 128), jnp.float32),
            pltpu.VMEM((C, 128), jnp.float32),
            pltpu.VMEM((C,), jnp.int32),
            pltpu.VMEM((C,), jnp.int32),
            pltpu.VMEM((C,), jnp.float32),
            pltpu.VMEM((C,), jnp.float32),
            pltpu.VMEM((H,), jnp.float32),
            pltpu.VMEM_SHARED((N_PAD, 128), jnp.float32),
            pltpu.SemaphoreType.DMA,
        ],
    )(l_pad, maxes, dsts, dstg, ea_pad, *m_tabs, wm, zeros)


def _sc_pair_body(p_hbm, q_hbm, ug_hbm, sg_hbm,
                  h1_hbm,
                  idxu_v, idxs_v, prows, qrows, semu, sems):
    wid = lax.axis_index("s") * NC + lax.axis_index("c")
    base = wid * EW

    def chunk_body(c, _):
        pltpu.sync_copy(ug_hbm.at[pl.ds(base + c * C, C)], idxu_v)
        pltpu.sync_copy(sg_hbm.at[pl.ds(base + c * C, C)], idxs_v)
        cpu_ = pltpu.async_copy(p_hbm.at[idxu_v], prows, semu)
        cps_ = pltpu.async_copy(q_hbm.at[idxs_v], qrows, sems)
        cpu_.wait()
        cps_.wait()

        def edge_body(e, _):
            for kc in range(16):
                x = prows[e, pl.ds(kc * L, L)] + qrows[e, pl.ds(kc * L, L)]
                prows[e, pl.ds(kc * L, L)] = 1.0 / (1.0 + jnp.exp(-x))
            return 0
        lax.fori_loop(0, C, edge_body, 0)
        pltpu.sync_copy(prows, h1_hbm.at[pl.ds(base + c * C, C)])
        return 0
    lax.fori_loop(0, EW // C, chunk_body, 0)


def _sc_pair_sigmoid(p_tab, q_tab, ug, sg):
    return pl.kernel(
        _sc_pair_body,
        out_type=jax.ShapeDtypeStruct((E_PAD, H), jnp.float32),
        mesh=plsc.VectorSubcoreMesh(core_axis_name="c", subcore_axis_name="s"),
        compiler_params=pltpu.CompilerParams(needs_layout_passes=False),
        scratch_types=[
            pltpu.VMEM((C,), jnp.int32),
            pltpu.VMEM((C,), jnp.int32),
            pltpu.VMEM((C, H), jnp.float32),
            pltpu.VMEM((C, H), jnp.float32),
            pltpu.SemaphoreType.DMA,
            pltpu.SemaphoreType.DMA,
        ],
    )(p_tab, q_tab, ug, sg)


def _sc_segmax_body(aw_hbm, seg_hbm, mx_hbm, awbuf, segbuf, maxacc):
    wid = lax.axis_index("s") * NC + lax.axis_index("c")
    base = wid * EW
    lane = lax.broadcasted_iota(jnp.int32, (L,), 0)
    pltpu.sync_copy(aw_hbm.at[pl.ds(base, EW)], awbuf)
    pltpu.sync_copy(seg_hbm.at[pl.ds(base, EW)], segbuf)

    def neg_init(i, _):
        maxacc[pl.ds(i * L, L)] = jnp.full((L,), -1e30, jnp.float32)
        return 0
    lax.fori_loop(0, N_PAD // L, neg_init, 0)

    def seg_max(gi, _):
        v = awbuf[pl.ds(gi * L, L)]
        k = segbuf[pl.ds(gi * L, L)]
        for r in range(1, L):
            ridx = (lane + r) & (L - 1)
            km = _vperm(k, ridx)
            vm = _vperm(v, ridx)
            v = jnp.where(km == k, jnp.maximum(v, vm), v)
        cur = plsc.load_gather(maxacc, [k])
        plsc.store_scatter(maxacc, [k], jnp.maximum(cur, v))
        return 0
    lax.fori_loop(0, EW // L, seg_max, 0)
    pltpu.sync_copy(maxacc, mx_hbm.at[pl.ds(wid * N_PAD, N_PAD)])


def _sc_segmax(aw, seg):
    return pl.kernel(
        _sc_segmax_body,
        out_type=jax.ShapeDtypeStruct((NW * N_PAD,), jnp.float32),
        mesh=plsc.VectorSubcoreMesh(core_axis_name="c", subcore_axis_name="s"),
        compiler_params=pltpu.CompilerParams(needs_layout_passes=False),
        scratch_types=[
            pltpu.VMEM((EW,), jnp.float32),
            pltpu.VMEM((EW,), jnp.int32),
            pltpu.VMEM((N_PAD,), jnp.float32),
        ],
    )(aw, seg)


def _sc_segexp_body(aw_hbm, seg_hbm, mx_hbm, ex_hbm, s_hbm,
                    awbuf, segbuf, m_buf, s_acc, mx_v):
    wid = lax.axis_index("s") * NC + lax.axis_index("c")
    base = wid * EW
    pltpu.sync_copy(aw_hbm.at[pl.ds(base, EW)], awbuf)
    pltpu.sync_copy(seg_hbm.at[pl.ds(base, EW)], segbuf)

    def mx_blk(blk, _):
        pltpu.sync_copy(mx_hbm.at[:, pl.ds(blk * 128, 128)], mx_v)

        def mx_grp(i, _):
            m = mx_v[0, pl.ds(i * L, L)]
            for w in range(1, NW):
                m = jnp.maximum(m, mx_v[w, pl.ds(i * L, L)])
            m_buf[pl.ds(blk * 128 + i * L, L)] = m
            return 0
        lax.fori_loop(0, 128 // L, mx_grp, 0)
        return 0
    lax.fori_loop(0, N_PAD // 128, mx_blk, 0)

    def z_init(i, _):
        s_acc[pl.ds(i * L, L)] = jnp.zeros((L,), jnp.float32)
        return 0
    lax.fori_loop(0, N_PAD // L, z_init, 0)

    def grp(gi, _):
        k = segbuf[pl.ds(gi * L, L)]
        mv = plsc.load_gather(m_buf, [k])
        ex = jnp.exp(awbuf[pl.ds(gi * L, L)] - mv)
        plsc.addupdate_scatter(s_acc, [k], ex)
        awbuf[pl.ds(gi * L, L)] = ex
        return 0
    lax.fori_loop(0, EW // L, grp, 0)
    pltpu.sync_copy(awbuf, ex_hbm.at[pl.ds(base, EW)])
    pltpu.sync_copy(s_acc, s_hbm.at[pl.ds(wid * N_PAD, N_PAD)])


def _sc_segexp(aw, seg, maxes):
    return pl.kernel(
        _sc_segexp_body,
        out_type=(jax.ShapeDtypeStruct((E_PAD,), jnp.float32),
                  jax.ShapeDtypeStruct((NW * N_PAD,), jnp.float32)),
        mesh=plsc.VectorSubcoreMesh(core_axis_name="c", subcore_axis_name="s"),
        compiler_params=pltpu.CompilerParams(needs_layout_passes=False),
        scratch_types=[
            pltpu.VMEM((EW,), jnp.float32),
            pltpu.VMEM((EW,), jnp.int32),
            pltpu.VMEM((N_PAD,), jnp.float32),
            pltpu.VMEM((N_PAD,), jnp.float32),
            pltpu.VMEM((NW, 128), jnp.float32),
        ],
    )(aw, seg, maxes)


def _sc_segdiv_body(ex_hbm, seg_hbm, sp_hbm, out_hbm,
                    exbuf, segbuf, s_buf, mx_v):
    wid = lax.axis_index("s") * NC + lax.axis_index("c")
    base = wid * EW
    pltpu.sync_copy(ex_hbm.at[pl.ds(base, EW)], exbuf)
    pltpu.sync_copy(seg_hbm.at[pl.ds(base, EW)], segbuf)

    def s_blk(blk, _):
        pltpu.sync_copy(sp_hbm.at[:, pl.ds(blk * 128, 128)], mx_v)

        def s_grp(i, _):
            m = mx_v[0, pl.ds(i * L, L)]
            for w in range(1, NW):
                m = m + mx_v[w, pl.ds(i * L, L)]
            s_buf[pl.ds(blk * 128 + i * L, L)] = m + 1e-16
            return 0
        lax.fori_loop(0, 128 // L, s_grp, 0)
        return 0
    lax.fori_loop(0, N_PAD // 128, s_blk, 0)

    def grp(gi, _):
        k = segbuf[pl.ds(gi * L, L)]
        sv = plsc.load_gather(s_buf, [k])
        exbuf[pl.ds(gi * L, L)] = exbuf[pl.ds(gi * L, L)] / sv
        return 0
    lax.fori_loop(0, EW // L, grp, 0)
    pltpu.sync_copy(exbuf, out_hbm.at[pl.ds(base, EW)])


def _sc_segdiv(ex, seg, s_part):
    return pl.kernel(
        _sc_segdiv_body,
        out_type=jax.ShapeDtypeStruct((E_PAD,), jnp.float32),
        mesh=plsc.VectorSubcoreMesh(core_axis_name="c", subcore_axis_name="s"),
        compiler_params=pltpu.CompilerParams(needs_layout_passes=False),
        scratch_types=[
            pltpu.VMEM((EW,), jnp.float32),
            pltpu.VMEM((EW,), jnp.int32),
            pltpu.VMEM((N_PAD,), jnp.float32),
            pltpu.VMEM((NW, 128), jnp.float32),
        ],
    )(ex, seg, s_part)


def _pad_edges(src, dst):
    pad = E_PAD - src.shape[0]
    src_g = jnp.concatenate([src, jnp.zeros((pad,), jnp.int32)])
    dst_g = jnp.concatenate([dst, jnp.zeros((pad,), jnp.int32)])
    dst_s = jnp.concatenate([dst, jnp.full((pad,), PAD_DST, jnp.int32)])
    return src_g, dst_g, dst_s


def _conv_rel(p, x_src, x_dst, src, dst, ea, n_dst, update_relu):
    w_att1 = p["att1"]["W"]
    w1a, w1b = w_att1[:, :H], w_att1[:, H:]
    b1 = p["att1"]["b"]
    wq, bq = p["Wq"]["W"], p["Wq"]["b"]
    wr, br = p["Wr"]["W"], p["Wr"]["b"]
    wa = jnp.dot(w1a, wq).T
    wb = jnp.dot(w1b, wr).T
    ba = jnp.dot(w1a, bq) + b1
    bb = jnp.dot(w1b, br)
    a_node = _mm(x_dst, wa, ba)
    b_node = _mm(x_src, wb, bb)

    w_msg = p["msg"]["W"]
    m_node = _mm(x_dst, w_msg[:, :H].T, p["msg"]["b"])
    wm_e = w_msg[:, H]

    w2 = p["att2"]["W"][0]

    src_g, dst_g, dst_s = _pad_edges(src, dst)
    l_pad, maxes = _sc_logits(a_node, b_node, dst_g, src_g, dst_s, w2)

    ea_pad = jnp.concatenate(
        [ea[:, 0], jnp.zeros((E_PAD - E,), jnp.float32)])
    m_tabs = [m_node[:, :128], m_node[:, 128:]]
    zeros = jnp.zeros((640, 128), jnp.float32)
    s_part, ag_part = _sc_aggr(l_pad, maxes.reshape(NW, N_PAD), dst_s, dst_g,
                               ea_pad, m_tabs, wm_e, zeros)
    s = s_part.reshape(NW, N_PAD).sum(axis=0)[:n_dst]
    ap = ag_part.reshape(2, NC, N_PAD, 128)
    aggr_raw = jnp.concatenate([ap[0, 0] + ap[0, 1], ap[1, 0] + ap[1, 1]],
                               axis=1)[:n_dst]
    aggr = aggr_raw / (s + 1e-16)[:, None]

    w_upd = p["upd"]["W"]
    upd = _mm(jnp.concatenate([aggr, x_dst], axis=1), w_upd.T, p["upd"]["b"])
    return jnp.maximum(upd, 0.0) if update_relu else upd


def kernel(x_user, x_server, edge_index_s2u, edge_index_u2u, edge_index_u2s,
           edge_attr_s2u, edge_attr_u2u, params):
    n_user = x_user.shape[0]
    s2u_src = edge_index_s2u[0].astype(jnp.int32)
    s2u_dst = edge_index_s2u[1].astype(jnp.int32)
    u2u_src = edge_index_u2u[0].astype(jnp.int32)
    u2u_dst = edge_index_u2u[1].astype(jnp.int32)
    u2s_u = edge_index_u2s[0].astype(jnp.int32)
    u2s_s = edge_index_u2s[1].astype(jnp.int32)

    ue, se = params["user_enc"], params["server_enc"]
    xu = _mm(x_user, ue[0]["W"].T, ue[0]["b"], act="relu")
    xu = _mm(xu, ue[1]["W"].T, ue[1]["b"], act="leaky")
    xs = _mm(x_server, se[0]["W"].T, se[0]["b"], act="relu")
    xs = _mm(xs, se[1]["W"].T, se[1]["b"], act="leaky")

    for cp in params["convs"]:
        u1 = _conv_rel(cp["s2u"], xs, xu, s2u_src, s2u_dst, edge_attr_s2u,
                       n_user, True)
        u2 = _conv_rel(cp["u2u"], xu, xu, u2u_src, u2u_dst, edge_attr_u2u,
                       n_user, False)
        xu = u1 + u2

    off = params["off"]
    o1 = _mm(xu, off[0]["W"].T, off[0]["b"], act="sigmoid")
    ol = _mm(o1, off[1]["W"].T, off[1]["b"])
    probs = jax.nn.softmax(ol, axis=-1)

    sel = params["sel"]
    w0 = sel[0]["W"]
    p_user = (_mm(xu, w0[:, :H].T, sel[0]["b"])
              + probs @ w0[:, H:H + 2].T)
    q_srv = _mm(xs, w0[:, H + 2:].T, jnp.zeros((H,), jnp.float32))

    pad = E_PAD - E
    ug = jnp.concatenate([u2s_u, jnp.zeros((pad,), jnp.int32)])
    sg = jnp.concatenate([u2s_s, jnp.zeros((pad,), jnp.int32)])
    us = jnp.concatenate([u2s_u, jnp.full((pad,), PAD_DST, jnp.int32)])
    h1 = _sc_pair_sigmoid(p_user, q_srv, ug, sg)
    h2 = _mm(h1, sel[1]["W"].T, sel[1]["b"], act="sigmoid")
    aw = _mm(h2, sel[2]["W"].T, sel[2]["b"])[:, 0]
    mx2 = _sc_segmax(aw, us)
    ex2, s2 = _sc_segexp(aw, us, mx2.reshape(NW, N_PAD))
    scheme = _sc_segdiv(ex2, us, s2.reshape(NW, N_PAD))[:E]
    return probs, scheme


# speedup vs baseline: 2.0585x; 1.0693x over previous
import functools

import jax
import jax.numpy as jnp
from jax import lax
from jax.experimental import pallas as pl
from jax.experimental.pallas import tpu as pltpu
from jax.experimental.pallas import tpu_sc as plsc

H = 256
ALPHA = 0.02

NC, NS, L = 2, 16, 16
NW = NC * NS
E = 160000
EW = 5120
E_PAD = EW * NW
C = 128
CL = 64
N_PAD = 10240
PAD_DST = N_PAD - 1
QH = 64


def _vperm(x, idx):
    return lax.gather(
        x, idx[:, None],
        lax.GatherDimensionNumbers(offset_dims=(), collapsed_slice_dims=(0,),
                                   start_index_map=(0,)),
        slice_sizes=(1,), mode=lax.GatherScatterMode.PROMISE_IN_BOUNDS)


def _sc_logits_body(a_hbm, b_hbm, dstg_hbm, srcg_hbm, dsts_hbm, w2_hbm,
                    l_hbm, mx_hbm,
                    dstg_v, srcg_v, dsts_v, w2_v,
                    arows0, brows0, arows1, brows1, lbuf, maxacc, sema, semb):
    wid = lax.axis_index("s") * NC + lax.axis_index("c")
    base = wid * EW
    pltpu.sync_copy(dstg_hbm.at[pl.ds(base, EW)], dstg_v)
    pltpu.sync_copy(srcg_hbm.at[pl.ds(base, EW)], srcg_v)
    pltpu.sync_copy(dsts_hbm.at[pl.ds(base, EW)], dsts_v)
    pltpu.sync_copy(w2_hbm, w2_v)
    w2c = [w2_v[pl.ds(16 * i, 16)] for i in range(16)]
    lane = lax.broadcasted_iota(jnp.int32, (L,), 0)
    rots = [(lane + r) & (L - 1) for r in (8, 4, 2, 1)]

    def neg_init(i, _):
        maxacc[pl.ds(i * L, L)] = jnp.full((L,), -1e30, jnp.float32)
        return 0
    lax.fori_loop(0, N_PAD // L, neg_init, 0)

    def issue(c, arows, brows):
        pltpu.async_copy(a_hbm.at[dstg_v.at[pl.ds(c * CL, CL)]], arows, sema)
        pltpu.async_copy(b_hbm.at[srcg_v.at[pl.ds(c * CL, CL)]], brows, semb)

    def drain(arows, brows):
        pltpu.make_async_copy(a_hbm.at[pl.ds(0, CL)], arows, sema).wait()
        pltpu.make_async_copy(b_hbm.at[pl.ds(0, CL)], brows, semb).wait()

    def compute(c, arows, brows):
        def edge_body(e, lvec):
            acc = jnp.zeros((L,), jnp.float32)
            for kc in range(16):
                va = arows[e, pl.ds(kc * 16, 16)]
                vb = brows[e, pl.ds(kc * 16, 16)]
                g = va + vb
                g = jnp.where(g >= 0.0, g, ALPHA * g)
                acc = acc + g * w2c[kc]
            for r in rots:
                acc = acc + _vperm(acc, r)
            lvec = lvec + jnp.where(lane == (e % L), acc, 0.0)

            @pl.when(e % L == L - 1)
            def _():
                lbuf[pl.ds(c * CL + (e // L) * L, L)] = lvec
            return jnp.where(e % L == L - 1, jnp.zeros((L,), jnp.float32),
                             lvec)
        lax.fori_loop(0, CL, edge_body, jnp.zeros((L,), jnp.float32))

    nch2 = EW // CL // 2
    issue(0, arows0, brows0)

    def chunk2_body(c2, _):
        a = 2 * c2
        issue(a + 1, arows1, brows1)
        drain(arows0, brows0)
        compute(a, arows0, brows0)

        @pl.when(c2 + 1 < nch2)
        def _():
            issue(a + 2, arows0, brows0)
        drain(arows1, brows1)
        compute(a + 1, arows1, brows1)
        return 0
    lax.fori_loop(0, nch2, chunk2_body, 0)

    def seg_max(gi, _):
        lvec = lbuf[pl.ds(gi * L, L)]
        dstv = dsts_v[pl.ds(gi * L, L)]
        k, v = dstv, lvec
        for r in range(1, L):
            ridx = (lane + r) & (L - 1)
            km = _vperm(k, ridx)
            vm = _vperm(v, ridx)
            v = jnp.where(km == k, jnp.maximum(v, vm), v)
        cur = plsc.load_gather(maxacc, [k])
        plsc.store_scatter(maxacc, [k], jnp.maximum(cur, v))
        return 0
    lax.fori_loop(0, EW // L, seg_max, 0)

    pltpu.sync_copy(lbuf, l_hbm.at[pl.ds(base, EW)])
    pltpu.sync_copy(maxacc, mx_hbm.at[pl.ds(wid * N_PAD, N_PAD)])


def _sc_logits(a_tab, b_tab, dstg, srcg, dsts, w2):
    return pl.kernel(
        _sc_logits_body,
        out_type=(jax.ShapeDtypeStruct((E_PAD,), jnp.float32),
                  jax.ShapeDtypeStruct((NW * N_PAD,), jnp.float32)),
        mesh=plsc.VectorSubcoreMesh(core_axis_name="c", subcore_axis_name="s"),
        compiler_params=pltpu.CompilerParams(needs_layout_passes=False),
        scratch_types=[
            pltpu.VMEM((EW,), jnp.int32),
            pltpu.VMEM((EW,), jnp.int32),
            pltpu.VMEM((EW,), jnp.int32),
            pltpu.VMEM((H,), jnp.float32),
            pltpu.VMEM((CL, H), jnp.float32),
            pltpu.VMEM((CL, H), jnp.float32),
            pltpu.VMEM((CL, H), jnp.float32),
            pltpu.VMEM((CL, H), jnp.float32),
            pltpu.VMEM((EW,), jnp.float32),
            pltpu.VMEM((N_PAD,), jnp.float32),
            pltpu.SemaphoreType.DMA,
            pltpu.SemaphoreType.DMA,
        ],
    )(a_tab, b_tab, dstg, srcg, dsts, w2)


def _mm_body(x_ref, w_ref, b_ref, o_ref, *, act):
    acc = jnp.dot(x_ref[...], w_ref[...], preferred_element_type=jnp.float32)
    acc = acc + b_ref[...]
    if act == "relu":
        acc = jnp.maximum(acc, 0.0)
    elif act == "leaky":
        acc = jnp.where(acc >= 0.0, acc, ALPHA * acc)
    elif act == "sigmoid":
        acc = jax.nn.sigmoid(acc)
    o_ref[...] = acc


@functools.partial(jax.jit, static_argnames=("act", "bn"))
def _mm(x, wt, b, act="none", bn=1024):
    n, k = x.shape
    d = wt.shape[1]
    bn = min(bn, n)
    grid = (pl.cdiv(n, bn),)
    return pl.pallas_call(
        functools.partial(_mm_body, act=act),
        grid=grid,
        in_specs=[
            pl.BlockSpec((bn, k), lambda i: (i, 0)),
            pl.BlockSpec((k, d), lambda i: (0, 0)),
            pl.BlockSpec((1, d), lambda i: (0, 0)),
        ],
        out_specs=pl.BlockSpec((bn, d), lambda i: (i, 0)),
        out_shape=jax.ShapeDtypeStruct((n, d), jnp.float32),
    )(x, wt, b.reshape(1, d))


def _seg_softmax(logits, seg, num_segments):
    m = jax.ops.segment_max(logits, seg, num_segments)
    m = jnp.where(jnp.isfinite(m), m, 0.0)
    ex = jnp.exp(logits - m[seg])
    s = jax.ops.segment_sum(ex, seg, num_segments)
    return ex / (s[seg] + 1e-16)


def _sc_aggr_body(l_hbm, mx_hbm, dsts_hbm, dstg_hbm, ea_hbm, m0_hbm, m1_hbm,
                  wm_hbm, zeros_hbm,
                  s_hbm, ag_hbm,
                  m_buf, s_acc, mx_v, m_rows, idxw_v, idxg_v, lc_v, eac_v,
                  wm_v, acc_sp, sema):
    cid = lax.axis_index("c")
    sid = lax.axis_index("s")
    wid = sid * NC + cid
    base = wid * EW

    pltpu.sync_copy(wm_hbm, wm_v)

    def mx_blk(blk, _):
        pltpu.sync_copy(mx_hbm.at[:, pl.ds(blk * 128, 128)], mx_v)

        def mx_grp(i, _):
            m = mx_v[0, pl.ds(i * L, L)]
            for w in range(1, NW):
                m = jnp.maximum(m, mx_v[w, pl.ds(i * L, L)])
            m_buf[pl.ds(blk * 128 + i * L, L)] = m
            return 0
        lax.fori_loop(0, 128 // L, mx_grp, 0)
        return 0
    lax.fori_loop(0, N_PAD // 128, mx_blk, 0)

    def z_init(i, _):
        s_acc[pl.ds(i * L, L)] = jnp.zeros((L,), jnp.float32)
        return 0
    lax.fori_loop(0, N_PAD // L, z_init, 0)

    for h in range(2):
        m_tab = m0_hbm if h == 0 else m1_hbm
        pltpu.sync_copy(zeros_hbm, acc_sp.at[pl.ds(sid * 640, 640)])
        plsc.subcore_barrier()

        def chunk_body(c, _):
            pltpu.sync_copy(dsts_hbm.at[pl.ds(base + c * C, C)], idxw_v)
            pltpu.sync_copy(dstg_hbm.at[pl.ds(base + c * C, C)], idxg_v)
            pltpu.sync_copy(l_hbm.at[pl.ds(base + c * C, C)], lc_v)
            pltpu.sync_copy(ea_hbm.at[pl.ds(base + c * C, C)], eac_v)
            pltpu.async_copy(m_tab.at[idxg_v], m_rows, sema).wait()

            def grp_body(g, _):
                dv = idxw_v[pl.ds(g * L, L)]
                lv = lc_v[pl.ds(g * L, L)]
                mv = plsc.load_gather(m_buf, [dv])
                ex = jnp.exp(lv - mv)
                if h == 0:
                    plsc.addupdate_scatter(s_acc, [dv], ex)
                eag = eac_v[pl.ds(g * L, L)]
                for el in range(L):
                    e = g * L + el
                    sel = jnp.full((L,), el, jnp.int32)
                    eav = _vperm(eag, sel)
                    exv = _vperm(ex, sel)
                    for kc in range(128 // L):
                        mr = m_rows[e, pl.ds(kc * L, L)]
                        wmv = wm_v[pl.ds(h * 128 + kc * L, L)]
                        m_rows[e, pl.ds(kc * L, L)] = (
                            jnp.maximum(mr + eav * wmv, 0.0) * exv)
                return 0
            lax.fori_loop(0, C // L, grp_body, 0)
            pltpu.sync_copy(m_rows, acc_sp.at[idxw_v], add=True)
            return 0
        lax.fori_loop(0, EW // C, chunk_body, 0)
        plsc.subcore_barrier()
        pltpu.sync_copy(
            acc_sp.at[pl.ds(sid * 640, 640)],
            ag_hbm.at[pl.ds((h * NC + cid) * N_PAD + sid * 640, 640)])
        plsc.subcore_barrier()
    pltpu.sync_copy(s_acc, s_hbm.at[pl.ds(wid * N_PAD, N_PAD)])


def _sc_aggr(l_pad, maxes, dsts, dstg, ea_pad, m_tabs, wm, zeros):
    return pl.kernel(
        _sc_aggr_body,
        out_type=(jax.ShapeDtypeStruct((NW * N_PAD,), jnp.float32),
                  jax.ShapeDtypeStruct((2 * NC * N_PAD, 128), jnp.float32)),
        mesh=plsc.VectorSubcoreMesh(core_axis_name="c", subcore_axis_name="s"),
        compiler_params=pltpu.CompilerParams(needs_layout_passes=False),
        scratch_types=[
            pltpu.VMEM((N_PAD,), jnp.float32),
            pltpu.VMEM((N_PAD,), jnp.float32),
            pltpu.VMEM((NW, 128), jnp.float32),
            pltpu.VMEM((C, 128), jnp.float32),
            pltpu.VMEM((C,), jnp.int32),
            pltpu.VMEM((C,), jnp.int32),
            pltpu.VMEM((C,), jnp.float32),
            pltpu.VMEM((C,), jnp.float32),
            pltpu.VMEM((H,), jnp.float32),
            pltpu.VMEM_SHARED((N_PAD, 128), jnp.float32),
            pltpu.SemaphoreType.DMA,
        ],
    )(l_pad, maxes, dsts, dstg, ea_pad, *m_tabs, wm, zeros)


def _sc_pair_body(p_hbm, q_hbm, ug_hbm, sg_hbm,
                  h1_hbm,
                  idxu_v, idxs_v, prows, qrows, semu, sems):
    wid = lax.axis_index("s") * NC + lax.axis_index("c")
    base = wid * EW

    def chunk_body(c, _):
        pltpu.sync_copy(ug_hbm.at[pl.ds(base + c * C, C)], idxu_v)
        pltpu.sync_copy(sg_hbm.at[pl.ds(base + c * C, C)], idxs_v)
        cpu_ = pltpu.async_copy(p_hbm.at[idxu_v], prows, semu)
        cps_ = pltpu.async_copy(q_hbm.at[idxs_v], qrows, sems)
        cpu_.wait()
        cps_.wait()

        def edge_body(e, _):
            for kc in range(16):
                x = prows[e, pl.ds(kc * L, L)] + qrows[e, pl.ds(kc * L, L)]
                prows[e, pl.ds(kc * L, L)] = 1.0 / (1.0 + jnp.exp(-x))
            return 0
        lax.fori_loop(0, C, edge_body, 0)
        pltpu.sync_copy(prows, h1_hbm.at[pl.ds(base + c * C, C)])
        return 0
    lax.fori_loop(0, EW // C, chunk_body, 0)


def _sc_pair_sigmoid(p_tab, q_tab, ug, sg):
    return pl.kernel(
        _sc_pair_body,
        out_type=jax.ShapeDtypeStruct((E_PAD, H), jnp.float32),
        mesh=plsc.VectorSubcoreMesh(core_axis_name="c", subcore_axis_name="s"),
        compiler_params=pltpu.CompilerParams(needs_layout_passes=False),
        scratch_types=[
            pltpu.VMEM((C,), jnp.int32),
            pltpu.VMEM((C,), jnp.int32),
            pltpu.VMEM((C, H), jnp.float32),
            pltpu.VMEM((C, H), jnp.float32),
            pltpu.SemaphoreType.DMA,
            pltpu.SemaphoreType.DMA,
        ],
    )(p_tab, q_tab, ug, sg)


def _sc_segmax_body(aw_hbm, seg_hbm, mx_hbm, awbuf, segbuf, maxacc):
    wid = lax.axis_index("s") * NC + lax.axis_index("c")
    base = wid * EW
    lane = lax.broadcasted_iota(jnp.int32, (L,), 0)
    pltpu.sync_copy(aw_hbm.at[pl.ds(base, EW)], awbuf)
    pltpu.sync_copy(seg_hbm.at[pl.ds(base, EW)], segbuf)

    def neg_init(i, _):
        maxacc[pl.ds(i * L, L)] = jnp.full((L,), -1e30, jnp.float32)
        return 0
    lax.fori_loop(0, N_PAD // L, neg_init, 0)

    def seg_max(gi, _):
        v = awbuf[pl.ds(gi * L, L)]
        k = segbuf[pl.ds(gi * L, L)]
        for r in range(1, L):
            ridx = (lane + r) & (L - 1)
            km = _vperm(k, ridx)
            vm = _vperm(v, ridx)
            v = jnp.where(km == k, jnp.maximum(v, vm), v)
        cur = plsc.load_gather(maxacc, [k])
        plsc.store_scatter(maxacc, [k], jnp.maximum(cur, v))
        return 0
    lax.fori_loop(0, EW // L, seg_max, 0)
    pltpu.sync_copy(maxacc, mx_hbm.at[pl.ds(wid * N_PAD, N_PAD)])


def _sc_segmax(aw, seg):
    return pl.kernel(
        _sc_segmax_body,
        out_type=jax.ShapeDtypeStruct((NW * N_PAD,), jnp.float32),
        mesh=plsc.VectorSubcoreMesh(core_axis_name="c", subcore_axis_name="s"),
        compiler_params=pltpu.CompilerParams(needs_layout_passes=False),
        scratch_types=[
            pltpu.VMEM((EW,), jnp.float32),
            pltpu.VMEM((EW,), jnp.int32),
            pltpu.VMEM((N_PAD,), jnp.float32),
        ],
    )(aw, seg)


def _sc_segexp_body(aw_hbm, seg_hbm, mx_hbm, ex_hbm, s_hbm,
                    awbuf, segbuf, m_buf, s_acc, mx_v):
    wid = lax.axis_index("s") * NC + lax.axis_index("c")
    base = wid * EW
    pltpu.sync_copy(aw_hbm.at[pl.ds(base, EW)], awbuf)
    pltpu.sync_copy(seg_hbm.at[pl.ds(base, EW)], segbuf)

    def mx_blk(blk, _):
        pltpu.sync_copy(mx_hbm.at[:, pl.ds(blk * 128, 128)], mx_v)

        def mx_grp(i, _):
            m = mx_v[0, pl.ds(i * L, L)]
            for w in range(1, NW):
                m = jnp.maximum(m, mx_v[w, pl.ds(i * L, L)])
            m_buf[pl.ds(blk * 128 + i * L, L)] = m
            return 0
        lax.fori_loop(0, 128 // L, mx_grp, 0)
        return 0
    lax.fori_loop(0, N_PAD // 128, mx_blk, 0)

    def z_init(i, _):
        s_acc[pl.ds(i * L, L)] = jnp.zeros((L,), jnp.float32)
        return 0
    lax.fori_loop(0, N_PAD // L, z_init, 0)

    def grp(gi, _):
        k = segbuf[pl.ds(gi * L, L)]
        mv = plsc.load_gather(m_buf, [k])
        ex = jnp.exp(awbuf[pl.ds(gi * L, L)] - mv)
        plsc.addupdate_scatter(s_acc, [k], ex)
        awbuf[pl.ds(gi * L, L)] = ex
        return 0
    lax.fori_loop(0, EW // L, grp, 0)
    pltpu.sync_copy(awbuf, ex_hbm.at[pl.ds(base, EW)])
    pltpu.sync_copy(s_acc, s_hbm.at[pl.ds(wid * N_PAD, N_PAD)])


def _sc_segexp(aw, seg, maxes):
    return pl.kernel(
        _sc_segexp_body,
        out_type=(jax.ShapeDtypeStruct((E_PAD,), jnp.float32),
                  jax.ShapeDtypeStruct((NW * N_PAD,), jnp.float32)),
        mesh=plsc.VectorSubcoreMesh(core_axis_name="c", subcore_axis_name="s"),
        compiler_params=pltpu.CompilerParams(needs_layout_passes=False),
        scratch_types=[
            pltpu.VMEM((EW,), jnp.float32),
            pltpu.VMEM((EW,), jnp.int32),
            pltpu.VMEM((N_PAD,), jnp.float32),
            pltpu.VMEM((N_PAD,), jnp.float32),
            pltpu.VMEM((NW, 128), jnp.float32),
        ],
    )(aw, seg, maxes)


def _sc_segdiv_body(ex_hbm, seg_hbm, sp_hbm, out_hbm,
                    exbuf, segbuf, s_buf, mx_v):
    wid = lax.axis_index("s") * NC + lax.axis_index("c")
    base = wid * EW
    pltpu.sync_copy(ex_hbm.at[pl.ds(base, EW)], exbuf)
    pltpu.sync_copy(seg_hbm.at[pl.ds(base, EW)], segbuf)

    def s_blk(blk, _):
        pltpu.sync_copy(sp_hbm.at[:, pl.ds(blk * 128, 128)], mx_v)

        def s_grp(i, _):
            m = mx_v[0, pl.ds(i * L, L)]
            for w in range(1, NW):
                m = m + mx_v[w, pl.ds(i * L, L)]
            s_buf[pl.ds(blk * 128 + i * L, L)] = m + 1e-16
            return 0
        lax.fori_loop(0, 128 // L, s_grp, 0)
        return 0
    lax.fori_loop(0, N_PAD // 128, s_blk, 0)

    def grp(gi, _):
        k = segbuf[pl.ds(gi * L, L)]
        sv = plsc.load_gather(s_buf, [k])
        exbuf[pl.ds(gi * L, L)] = exbuf[pl.ds(gi * L, L)] / sv
        return 0
    lax.fori_loop(0, EW // L, grp, 0)
    pltpu.sync_copy(exbuf, out_hbm.at[pl.ds(base, EW)])


def _sc_segdiv(ex, seg, s_part):
    return pl.kernel(
        _sc_segdiv_body,
        out_type=jax.ShapeDtypeStruct((E_PAD,), jnp.float32),
        mesh=plsc.VectorSubcoreMesh(core_axis_name="c", subcore_axis_name="s"),
        compiler_params=pltpu.CompilerParams(needs_layout_passes=False),
        scratch_types=[
            pltpu.VMEM((EW,), jnp.float32),
            pltpu.VMEM((EW,), jnp.int32),
            pltpu.VMEM((N_PAD,), jnp.float32),
            pltpu.VMEM((NW, 128), jnp.float32),
        ],
    )(ex, seg, s_part)


def _pad_edges(src, dst):
    pad = E_PAD - src.shape[0]
    src_g = jnp.concatenate([src, jnp.zeros((pad,), jnp.int32)])
    dst_g = jnp.concatenate([dst, jnp.zeros((pad,), jnp.int32)])
    dst_s = jnp.concatenate([dst, jnp.full((pad,), PAD_DST, jnp.int32)])
    return src_g, dst_g, dst_s


def _conv_rel(p, x_src, x_dst, src, dst, ea, n_dst, update_relu):
    w_att1 = p["att1"]["W"]
    w1a, w1b = w_att1[:, :H], w_att1[:, H:]
    b1 = p["att1"]["b"]
    wq, bq = p["Wq"]["W"], p["Wq"]["b"]
    wr, br = p["Wr"]["W"], p["Wr"]["b"]
    wa = jnp.dot(w1a, wq).T
    wb = jnp.dot(w1b, wr).T
    ba = jnp.dot(w1a, bq) + b1
    bb = jnp.dot(w1b, br)
    a_node = _mm(x_dst, wa, ba)
    b_node = _mm(x_src, wb, bb)

    w_msg = p["msg"]["W"]
    m_node = _mm(x_dst, w_msg[:, :H].T, p["msg"]["b"])
    wm_e = w_msg[:, H]

    w2 = p["att2"]["W"][0]

    src_g, dst_g, dst_s = _pad_edges(src, dst)
    l_pad, maxes = _sc_logits(a_node, b_node, dst_g, src_g, dst_s, w2)

    ea_pad = jnp.concatenate(
        [ea[:, 0], jnp.zeros((E_PAD - E,), jnp.float32)])
    m_tabs = [m_node[:, :128], m_node[:, 128:]]
    zeros = jnp.zeros((640, 128), jnp.float32)
    s_part, ag_part = _sc_aggr(l_pad, maxes.reshape(NW, N_PAD), dst_s, dst_g,
                               ea_pad, m_tabs, wm_e, zeros)
    s = s_part.reshape(NW, N_PAD).sum(axis=0)[:n_dst]
    ap = ag_part.reshape(2, NC, N_PAD, 128)
    aggr_raw = jnp.concatenate([ap[0, 0] + ap[0, 1], ap[1, 0] + ap[1, 1]],
                               axis=1)[:n_dst]
    aggr = aggr_raw / (s + 1e-16)[:, None]

    w_upd = p["upd"]["W"]
    upd = _mm(jnp.concatenate([aggr, x_dst], axis=1), w_upd.T, p["upd"]["b"])
    return jnp.maximum(upd, 0.0) if update_relu else upd


def kernel(x_user, x_server, edge_index_s2u, edge_index_u2u, edge_index_u2s,
           edge_attr_s2u, edge_attr_u2u, params):
    n_user = x_user.shape[0]
    s2u_src = edge_index_s2u[0].astype(jnp.int32)
    s2u_dst = edge_index_s2u[1].astype(jnp.int32)
    u2u_src = edge_index_u2u[0].astype(jnp.int32)
    u2u_dst = edge_index_u2u[1].astype(jnp.int32)
    u2s_u = edge_index_u2s[0].astype(jnp.int32)
    u2s_s = edge_index_u2s[1].astype(jnp.int32)

    ue, se = params["user_enc"], params["server_enc"]
    xu = _mm(x_user, ue[0]["W"].T, ue[0]["b"], act="relu")
    xu = _mm(xu, ue[1]["W"].T, ue[1]["b"], act="leaky")
    xs = _mm(x_server, se[0]["W"].T, se[0]["b"], act="relu")
    xs = _mm(xs, se[1]["W"].T, se[1]["b"], act="leaky")

    for cp in params["convs"]:
        u1 = _conv_rel(cp["s2u"], xs, xu, s2u_src, s2u_dst, edge_attr_s2u,
                       n_user, True)
        u2 = _conv_rel(cp["u2u"], xu, xu, u2u_src, u2u_dst, edge_attr_u2u,
                       n_user, False)
        xu = u1 + u2

    off = params["off"]
    o1 = _mm(xu, off[0]["W"].T, off[0]["b"], act="sigmoid")
    ol = _mm(o1, off[1]["W"].T, off[1]["b"])
    probs = jax.nn.softmax(ol, axis=-1)

    sel = params["sel"]
    w0 = sel[0]["W"]
    p_user = (_mm(xu, w0[:, :H].T, sel[0]["b"])
              + probs @ w0[:, H:H + 2].T)
    q_srv = _mm(xs, w0[:, H + 2:].T, jnp.zeros((H,), jnp.float32))

    pad = E_PAD - E
    ug = jnp.concatenate([u2s_u, jnp.zeros((pad,), jnp.int32)])
    sg = jnp.concatenate([u2s_s, jnp.zeros((pad,), jnp.int32)])
    us = jnp.concatenate([u2s_u, jnp.full((pad,), PAD_DST, jnp.int32)])
    h1 = _sc_pair_sigmoid(p_user, q_srv, ug, sg)
    h2 = _mm(h1, sel[1]["W"].T, sel[1]["b"], act="sigmoid")
    aw = _mm(h2, sel[2]["W"].T, sel[2]["b"])[:, 0]
    mx2 = _sc_segmax(aw, us)
    ex2, s2 = _sc_segexp(aw, us, mx2.reshape(NW, N_PAD))
    scheme = _sc_segdiv(ex2, us, s2.reshape(NW, N_PAD))[:E]
    return probs, scheme


# speedup vs baseline: 2.3923x; 1.1622x over previous
import functools

import jax
import jax.numpy as jnp
from jax import lax
from jax.experimental import pallas as pl
from jax.experimental.pallas import tpu as pltpu
from jax.experimental.pallas import tpu_sc as plsc

H = 256
ALPHA = 0.02

NC, NS, L = 2, 16, 16
NW = NC * NS
E = 160000
EW = 5120
E_PAD = EW * NW
C = 128
CL = 64
N_PAD = 10240
PAD_DST = N_PAD - 1
QH = 64


def _vperm(x, idx):
    return lax.gather(
        x, idx[:, None],
        lax.GatherDimensionNumbers(offset_dims=(), collapsed_slice_dims=(0,),
                                   start_index_map=(0,)),
        slice_sizes=(1,), mode=lax.GatherScatterMode.PROMISE_IN_BOUNDS)


def _sc_logits_body(a_hbm, b_hbm, dstg_hbm, srcg_hbm, dsts_hbm, w2_hbm,
                    l_hbm, mx_hbm,
                    dstg_v, srcg_v, dsts_v, w2_v,
                    arows0, brows0, arows1, brows1, lbuf, maxacc, sema, semb):
    wid = lax.axis_index("s") * NC + lax.axis_index("c")
    base = wid * EW
    pltpu.sync_copy(dstg_hbm.at[pl.ds(base, EW)], dstg_v)
    pltpu.sync_copy(srcg_hbm.at[pl.ds(base, EW)], srcg_v)
    pltpu.sync_copy(dsts_hbm.at[pl.ds(base, EW)], dsts_v)
    pltpu.sync_copy(w2_hbm, w2_v)
    w2c = [w2_v[pl.ds(16 * i, 16)] for i in range(16)]
    lane = lax.broadcasted_iota(jnp.int32, (L,), 0)
    rots = [(lane + r) & (L - 1) for r in (8, 4, 2, 1)]

    def neg_init(i, _):
        maxacc[pl.ds(i * L, L)] = jnp.full((L,), -1e30, jnp.float32)
        return 0
    lax.fori_loop(0, N_PAD // L, neg_init, 0)

    def issue(c, arows, brows):
        pltpu.async_copy(a_hbm.at[dstg_v.at[pl.ds(c * CL, CL)]], arows, sema)
        pltpu.async_copy(b_hbm.at[srcg_v.at[pl.ds(c * CL, CL)]], brows, semb)

    def drain(arows, brows):
        pltpu.make_async_copy(a_hbm.at[pl.ds(0, CL)], arows, sema).wait()
        pltpu.make_async_copy(b_hbm.at[pl.ds(0, CL)], brows, semb).wait()

    def compute(c, arows, brows):
        def edge_body(e, lvec):
            acc = jnp.zeros((L,), jnp.float32)
            for kc in range(16):
                va = arows[e, pl.ds(kc * 16, 16)]
                vb = brows[e, pl.ds(kc * 16, 16)]
                g = va + vb
                g = jnp.where(g >= 0.0, g, ALPHA * g)
                acc = acc + g * w2c[kc]
            for r in rots:
                acc = acc + _vperm(acc, r)
            lvec = lvec + jnp.where(lane == (e % L), acc, 0.0)

            @pl.when(e % L == L - 1)
            def _():
                lbuf[pl.ds(c * CL + (e // L) * L, L)] = lvec
            return jnp.where(e % L == L - 1, jnp.zeros((L,), jnp.float32),
                             lvec)
        lax.fori_loop(0, CL, edge_body, jnp.zeros((L,), jnp.float32))

    nch2 = EW // CL // 2
    issue(0, arows0, brows0)

    def chunk2_body(c2, _):
        a = 2 * c2
        issue(a + 1, arows1, brows1)
        drain(arows0, brows0)
        compute(a, arows0, brows0)

        @pl.when(c2 + 1 < nch2)
        def _():
            issue(a + 2, arows0, brows0)
        drain(arows1, brows1)
        compute(a + 1, arows1, brows1)
        return 0
    lax.fori_loop(0, nch2, chunk2_body, 0)

    def seg_max(gi, _):
        lvec = lbuf[pl.ds(gi * L, L)]
        dstv = dsts_v[pl.ds(gi * L, L)]
        k, v = dstv, lvec
        for r in range(1, L):
            ridx = (lane + r) & (L - 1)
            km = _vperm(k, ridx)
            vm = _vperm(v, ridx)
            v = jnp.where(km == k, jnp.maximum(v, vm), v)
        cur = plsc.load_gather(maxacc, [k])
        plsc.store_scatter(maxacc, [k], jnp.maximum(cur, v))
        return 0
    lax.fori_loop(0, EW // L, seg_max, 0)

    pltpu.sync_copy(lbuf, l_hbm.at[pl.ds(base, EW)])
    pltpu.sync_copy(maxacc, mx_hbm.at[pl.ds(wid * N_PAD, N_PAD)])


def _sc_logits(a_tab, b_tab, dstg, srcg, dsts, w2):
    return pl.kernel(
        _sc_logits_body,
        out_type=(jax.ShapeDtypeStruct((E_PAD,), jnp.float32),
                  jax.ShapeDtypeStruct((NW * N_PAD,), jnp.float32)),
        mesh=plsc.VectorSubcoreMesh(core_axis_name="c", subcore_axis_name="s"),
        compiler_params=pltpu.CompilerParams(needs_layout_passes=False),
        scratch_types=[
            pltpu.VMEM((EW,), jnp.int32),
            pltpu.VMEM((EW,), jnp.int32),
            pltpu.VMEM((EW,), jnp.int32),
            pltpu.VMEM((H,), jnp.float32),
            pltpu.VMEM((CL, H), jnp.float32),
            pltpu.VMEM((CL, H), jnp.float32),
            pltpu.VMEM((CL, H), jnp.float32),
            pltpu.VMEM((CL, H), jnp.float32),
            pltpu.VMEM((EW,), jnp.float32),
            pltpu.VMEM((N_PAD,), jnp.float32),
            pltpu.SemaphoreType.DMA,
            pltpu.SemaphoreType.DMA,
        ],
    )(a_tab, b_tab, dstg, srcg, dsts, w2)


def _mm_body(x_ref, w_ref, b_ref, o_ref, *, act):
    acc = jnp.dot(x_ref[...], w_ref[...], preferred_element_type=jnp.float32)
    acc = acc + b_ref[...]
    if act == "relu":
        acc = jnp.maximum(acc, 0.0)
    elif act == "leaky":
        acc = jnp.where(acc >= 0.0, acc, ALPHA * acc)
    elif act == "sigmoid":
        acc = jax.nn.sigmoid(acc)
    o_ref[...] = acc


@functools.partial(jax.jit, static_argnames=("act", "bn"))
def _mm(x, wt, b, act="none", bn=1024):
    n, k = x.shape
    d = wt.shape[1]
    bn = min(bn, n)
    grid = (pl.cdiv(n, bn),)
    return pl.pallas_call(
        functools.partial(_mm_body, act=act),
        grid=grid,
        in_specs=[
            pl.BlockSpec((bn, k), lambda i: (i, 0)),
            pl.BlockSpec((k, d), lambda i: (0, 0)),
            pl.BlockSpec((1, d), lambda i: (0, 0)),
        ],
        out_specs=pl.BlockSpec((bn, d), lambda i: (i, 0)),
        out_shape=jax.ShapeDtypeStruct((n, d), jnp.float32),
    )(x, wt, b.reshape(1, d))


def _seg_softmax(logits, seg, num_segments):
    m = jax.ops.segment_max(logits, seg, num_segments)
    m = jnp.where(jnp.isfinite(m), m, 0.0)
    ex = jnp.exp(logits - m[seg])
    s = jax.ops.segment_sum(ex, seg, num_segments)
    return ex / (s[seg] + 1e-16)


def _sc_combine_body(part_hbm, out_hbm, mx_v, slice_v, *, is_sum):
    wid = lax.axis_index("s") * NC + lax.axis_index("c")
    nblk = N_PAD // 128

    def do_blk(blk):
        pltpu.sync_copy(part_hbm.at[:, pl.ds(blk * 128, 128)], mx_v)

        def grp(i, _):
            m = mx_v[0, pl.ds(i * L, L)]
            for w in range(1, NW):
                if is_sum:
                    m = m + mx_v[w, pl.ds(i * L, L)]
                else:
                    m = jnp.maximum(m, mx_v[w, pl.ds(i * L, L)])
            slice_v[pl.ds(i * L, L)] = m
            return 0
        lax.fori_loop(0, 128 // L, grp, 0)
        pltpu.sync_copy(slice_v, out_hbm.at[pl.ds(blk * 128, 128)])

    def blk_loop(i, _):
        blk = wid + i * NW

        @pl.when(blk < nblk)
        def _():
            do_blk(blk)
        return 0
    lax.fori_loop(0, (nblk + NW - 1) // NW, blk_loop, 0)


def _sc_combine(part, is_sum):
    return pl.kernel(
        functools.partial(_sc_combine_body, is_sum=is_sum),
        out_type=jax.ShapeDtypeStruct((N_PAD,), jnp.float32),
        mesh=plsc.VectorSubcoreMesh(core_axis_name="c", subcore_axis_name="s"),
        compiler_params=pltpu.CompilerParams(needs_layout_passes=False),
        scratch_types=[
            pltpu.VMEM((NW, 128), jnp.float32),
            pltpu.VMEM((128,), jnp.float32),
        ],
    )(part)


def _sc_aggr_body(l_hbm, m_hbm, dsts_hbm, dstg_hbm, ea_hbm, m0_hbm, m1_hbm,
                  wm_hbm, zeros_hbm,
                  s_hbm, ag_hbm,
                  m_buf, s_acc, dstg_v, lbuf, m_rows, idxw_v, eac_v,
                  wm_v, acc_sp, sema):
    cid = lax.axis_index("c")
    sid = lax.axis_index("s")
    wid = sid * NC + cid
    base = wid * EW

    pltpu.sync_copy(wm_hbm, wm_v)
    pltpu.sync_copy(m_hbm, m_buf)
    pltpu.sync_copy(dstg_hbm.at[pl.ds(base, EW)], dstg_v)
    pltpu.sync_copy(l_hbm.at[pl.ds(base, EW)], lbuf)

    def z_init(i, _):
        s_acc[pl.ds(i * L, L)] = jnp.zeros((L,), jnp.float32)
        return 0
    lax.fori_loop(0, N_PAD // L, z_init, 0)

    for h in range(2):
        m_tab = m0_hbm if h == 0 else m1_hbm
        pltpu.sync_copy(zeros_hbm, acc_sp.at[pl.ds(sid * 640, 640)])
        plsc.subcore_barrier()

        def chunk_body(c, _):
            pltpu.sync_copy(dsts_hbm.at[pl.ds(base + c * C, C)], idxw_v)
            pltpu.sync_copy(ea_hbm.at[pl.ds(base + c * C, C)], eac_v)
            pltpu.async_copy(
                m_tab.at[dstg_v.at[pl.ds(c * C, C)]], m_rows, sema).wait()

            def grp_body(g, _):
                dv = idxw_v[pl.ds(g * L, L)]
                lv = lbuf[pl.ds(c * C + g * L, L)]
                mv = plsc.load_gather(m_buf, [dv])
                ex = jnp.exp(lv - mv)
                if h == 0:
                    plsc.addupdate_scatter(s_acc, [dv], ex)
                eag = eac_v[pl.ds(g * L, L)]
                for el in range(L):
                    e = g * L + el
                    sel = jnp.full((L,), el, jnp.int32)
                    eav = _vperm(eag, sel)
                    exv = _vperm(ex, sel)
                    for kc in range(128 // L):
                        mr = m_rows[e, pl.ds(kc * L, L)]
                        wmv = wm_v[pl.ds(h * 128 + kc * L, L)]
                        m_rows[e, pl.ds(kc * L, L)] = (
                            jnp.maximum(mr + eav * wmv, 0.0) * exv)
                return 0
            lax.fori_loop(0, C // L, grp_body, 0)
            pltpu.sync_copy(m_rows, acc_sp.at[idxw_v], add=True)
            return 0
        lax.fori_loop(0, EW // C, chunk_body, 0)
        plsc.subcore_barrier()
        pltpu.sync_copy(
            acc_sp.at[pl.ds(sid * 640, 640)],
            ag_hbm.at[pl.ds((h * NC + cid) * N_PAD + sid * 640, 640)])
        plsc.subcore_barrier()
    pltpu.sync_copy(s_acc, s_hbm.at[pl.ds(wid * N_PAD, N_PAD)])


def _sc_aggr(l_pad, m_comb, dsts, dstg, ea_pad, m_tabs, wm, zeros):
    return pl.kernel(
        _sc_aggr_body,
        out_type=(jax.ShapeDtypeStruct((NW * N_PAD,), jnp.float32),
                  jax.ShapeDtypeStruct((2 * NC * N_PAD, 128), jnp.float32)),
        mesh=plsc.VectorSubcoreMesh(core_axis_name="c", subcore_axis_name="s"),
        compiler_params=pltpu.CompilerParams(needs_layout_passes=False),
        scratch_types=[
            pltpu.VMEM((N_PAD,), jnp.float32),
            pltpu.VMEM((N_PAD,), jnp.float32),
            pltpu.VMEM((EW,), jnp.int32),
            pltpu.VMEM((EW,), jnp.float32),
            pltpu.VMEM((C, 128), jnp.float32),
            pltpu.VMEM((C,), jnp.int32),
            pltpu.VMEM((C,), jnp.float32),
            pltpu.VMEM((H,), jnp.float32),
            pltpu.VMEM_SHARED((N_PAD, 128), jnp.float32),
            pltpu.SemaphoreType.DMA,
        ],
    )(l_pad, m_comb, dsts, dstg, ea_pad, *m_tabs, wm, zeros)


def _sc_pair_body(p_hbm, q_hbm, ug_hbm, sg_hbm,
                  h1_hbm,
                  idxu_v, idxs_v, prows, qrows, semu, sems):
    wid = lax.axis_index("s") * NC + lax.axis_index("c")
    base = wid * EW

    def chunk_body(c, _):
        pltpu.sync_copy(ug_hbm.at[pl.ds(base + c * C, C)], idxu_v)
        pltpu.sync_copy(sg_hbm.at[pl.ds(base + c * C, C)], idxs_v)
        cpu_ = pltpu.async_copy(p_hbm.at[idxu_v], prows, semu)
        cps_ = pltpu.async_copy(q_hbm.at[idxs_v], qrows, sems)
        cpu_.wait()
        cps_.wait()

        def edge_body(e, _):
            for kc in range(16):
                x = prows[e, pl.ds(kc * L, L)] + qrows[e, pl.ds(kc * L, L)]
                prows[e, pl.ds(kc * L, L)] = 1.0 / (1.0 + jnp.exp(-x))
            return 0
        lax.fori_loop(0, C, edge_body, 0)
        pltpu.sync_copy(prows, h1_hbm.at[pl.ds(base + c * C, C)])
        return 0
    lax.fori_loop(0, EW // C, chunk_body, 0)


def _sc_pair_sigmoid(p_tab, q_tab, ug, sg):
    return pl.kernel(
        _sc_pair_body,
        out_type=jax.ShapeDtypeStruct((E_PAD, H), jnp.float32),
        mesh=plsc.VectorSubcoreMesh(core_axis_name="c", subcore_axis_name="s"),
        compiler_params=pltpu.CompilerParams(needs_layout_passes=False),
        scratch_types=[
            pltpu.VMEM((C,), jnp.int32),
            pltpu.VMEM((C,), jnp.int32),
            pltpu.VMEM((C, H), jnp.float32),
            pltpu.VMEM((C, H), jnp.float32),
            pltpu.SemaphoreType.DMA,
            pltpu.SemaphoreType.DMA,
        ],
    )(p_tab, q_tab, ug, sg)


def _sc_segmax_body(aw_hbm, seg_hbm, mx_hbm, awbuf, segbuf, maxacc):
    wid = lax.axis_index("s") * NC + lax.axis_index("c")
    base = wid * EW
    lane = lax.broadcasted_iota(jnp.int32, (L,), 0)
    pltpu.sync_copy(aw_hbm.at[pl.ds(base, EW)], awbuf)
    pltpu.sync_copy(seg_hbm.at[pl.ds(base, EW)], segbuf)

    def neg_init(i, _):
        maxacc[pl.ds(i * L, L)] = jnp.full((L,), -1e30, jnp.float32)
        return 0
    lax.fori_loop(0, N_PAD // L, neg_init, 0)

    def seg_max(gi, _):
        v = awbuf[pl.ds(gi * L, L)]
        k = segbuf[pl.ds(gi * L, L)]
        for r in range(1, L):
            ridx = (lane + r) & (L - 1)
            km = _vperm(k, ridx)
            vm = _vperm(v, ridx)
            v = jnp.where(km == k, jnp.maximum(v, vm), v)
        cur = plsc.load_gather(maxacc, [k])
        plsc.store_scatter(maxacc, [k], jnp.maximum(cur, v))
        return 0
    lax.fori_loop(0, EW // L, seg_max, 0)
    pltpu.sync_copy(maxacc, mx_hbm.at[pl.ds(wid * N_PAD, N_PAD)])


def _sc_segmax(aw, seg):
    return pl.kernel(
        _sc_segmax_body,
        out_type=jax.ShapeDtypeStruct((NW * N_PAD,), jnp.float32),
        mesh=plsc.VectorSubcoreMesh(core_axis_name="c", subcore_axis_name="s"),
        compiler_params=pltpu.CompilerParams(needs_layout_passes=False),
        scratch_types=[
            pltpu.VMEM((EW,), jnp.float32),
            pltpu.VMEM((EW,), jnp.int32),
            pltpu.VMEM((N_PAD,), jnp.float32),
        ],
    )(aw, seg)


def _sc_segexp_body(aw_hbm, seg_hbm, m_hbm, ex_hbm, s_hbm,
                    awbuf, segbuf, m_buf, s_acc):
    wid = lax.axis_index("s") * NC + lax.axis_index("c")
    base = wid * EW
    pltpu.sync_copy(aw_hbm.at[pl.ds(base, EW)], awbuf)
    pltpu.sync_copy(seg_hbm.at[pl.ds(base, EW)], segbuf)
    pltpu.sync_copy(m_hbm, m_buf)

    def z_init(i, _):
        s_acc[pl.ds(i * L, L)] = jnp.zeros((L,), jnp.float32)
        return 0
    lax.fori_loop(0, N_PAD // L, z_init, 0)

    def grp(gi, _):
        k = segbuf[pl.ds(gi * L, L)]
        mv = plsc.load_gather(m_buf, [k])
        ex = jnp.exp(awbuf[pl.ds(gi * L, L)] - mv)
        plsc.addupdate_scatter(s_acc, [k], ex)
        awbuf[pl.ds(gi * L, L)] = ex
        return 0
    lax.fori_loop(0, EW // L, grp, 0)
    pltpu.sync_copy(awbuf, ex_hbm.at[pl.ds(base, EW)])
    pltpu.sync_copy(s_acc, s_hbm.at[pl.ds(wid * N_PAD, N_PAD)])


def _sc_segexp(aw, seg, m_comb):
    return pl.kernel(
        _sc_segexp_body,
        out_type=(jax.ShapeDtypeStruct((E_PAD,), jnp.float32),
                  jax.ShapeDtypeStruct((NW * N_PAD,), jnp.float32)),
        mesh=plsc.VectorSubcoreMesh(core_axis_name="c", subcore_axis_name="s"),
        compiler_params=pltpu.CompilerParams(needs_layout_passes=False),
        scratch_types=[
            pltpu.VMEM((EW,), jnp.float32),
            pltpu.VMEM((EW,), jnp.int32),
            pltpu.VMEM((N_PAD,), jnp.float32),
            pltpu.VMEM((N_PAD,), jnp.float32),
        ],
    )(aw, seg, m_comb)


def _sc_segdiv_body(ex_hbm, seg_hbm, s_hbm, out_hbm,
                    exbuf, segbuf, s_buf):
    wid = lax.axis_index("s") * NC + lax.axis_index("c")
    base = wid * EW
    pltpu.sync_copy(ex_hbm.at[pl.ds(base, EW)], exbuf)
    pltpu.sync_copy(seg_hbm.at[pl.ds(base, EW)], segbuf)
    pltpu.sync_copy(s_hbm, s_buf)

    def grp(gi, _):
        k = segbuf[pl.ds(gi * L, L)]
        sv = plsc.load_gather(s_buf, [k])
        exbuf[pl.ds(gi * L, L)] = exbuf[pl.ds(gi * L, L)] / (sv + 1e-16)
        return 0
    lax.fori_loop(0, EW // L, grp, 0)
    pltpu.sync_copy(exbuf, out_hbm.at[pl.ds(base, EW)])


def _sc_segdiv(ex, seg, s_comb):
    return pl.kernel(
        _sc_segdiv_body,
        out_type=jax.ShapeDtypeStruct((E_PAD,), jnp.float32),
        mesh=plsc.VectorSubcoreMesh(core_axis_name="c", subcore_axis_name="s"),
        compiler_params=pltpu.CompilerParams(needs_layout_passes=False),
        scratch_types=[
            pltpu.VMEM((EW,), jnp.float32),
            pltpu.VMEM((EW,), jnp.int32),
            pltpu.VMEM((N_PAD,), jnp.float32),
        ],
    )(ex, seg, s_comb)


def _pad_edges(src, dst):
    pad = E_PAD - src.shape[0]
    src_g = jnp.concatenate([src, jnp.zeros((pad,), jnp.int32)])
    dst_g = jnp.concatenate([dst, jnp.zeros((pad,), jnp.int32)])
    dst_s = jnp.concatenate([dst, jnp.full((pad,), PAD_DST, jnp.int32)])
    return src_g, dst_g, dst_s


def _conv_rel(p, x_src, x_dst, src, dst, ea, n_dst, update_relu):
    w_att1 = p["att1"]["W"]
    w1a, w1b = w_att1[:, :H], w_att1[:, H:]
    b1 = p["att1"]["b"]
    wq, bq = p["Wq"]["W"], p["Wq"]["b"]
    wr, br = p["Wr"]["W"], p["Wr"]["b"]
    wa = jnp.dot(w1a, wq).T
    wb = jnp.dot(w1b, wr).T
    ba = jnp.dot(w1a, bq) + b1
    bb = jnp.dot(w1b, br)
    a_node = _mm(x_dst, wa, ba)
    b_node = _mm(x_src, wb, bb)

    w_msg = p["msg"]["W"]
    m_node = _mm(x_dst, w_msg[:, :H].T, p["msg"]["b"])
    wm_e = w_msg[:, H]

    w2 = p["att2"]["W"][0]

    src_g, dst_g, dst_s = _pad_edges(src, dst)
    l_pad, maxes = _sc_logits(a_node, b_node, dst_g, src_g, dst_s, w2)

    ea_pad = jnp.concatenate(
        [ea[:, 0], jnp.zeros((E_PAD - E,), jnp.float32)])
    m_tabs = [m_node[:, :128], m_node[:, 128:]]
    zeros = jnp.zeros((640, 128), jnp.float32)
    m_comb = _sc_combine(maxes.reshape(NW, N_PAD), False)
    s_part, ag_part = _sc_aggr(l_pad, m_comb, dst_s, dst_g,
                               ea_pad, m_tabs, wm_e, zeros)
    s = s_part.reshape(NW, N_PAD).sum(axis=0)[:n_dst]
    ap = ag_part.reshape(2, NC, N_PAD, 128)
    aggr_raw = jnp.concatenate([ap[0, 0] + ap[0, 1], ap[1, 0] + ap[1, 1]],
                               axis=1)[:n_dst]
    aggr = aggr_raw / (s + 1e-16)[:, None]

    w_upd = p["upd"]["W"]
    upd = _mm(jnp.concatenate([aggr, x_dst], axis=1), w_upd.T, p["upd"]["b"])
    return jnp.maximum(upd, 0.0) if update_relu else upd


def kernel(x_user, x_server, edge_index_s2u, edge_index_u2u, edge_index_u2s,
           edge_attr_s2u, edge_attr_u2u, params):
    n_user = x_user.shape[0]
    s2u_src = edge_index_s2u[0].astype(jnp.int32)
    s2u_dst = edge_index_s2u[1].astype(jnp.int32)
    u2u_src = edge_index_u2u[0].astype(jnp.int32)
    u2u_dst = edge_index_u2u[1].astype(jnp.int32)
    u2s_u = edge_index_u2s[0].astype(jnp.int32)
    u2s_s = edge_index_u2s[1].astype(jnp.int32)

    ue, se = params["user_enc"], params["server_enc"]
    xu = _mm(x_user, ue[0]["W"].T, ue[0]["b"], act="relu")
    xu = _mm(xu, ue[1]["W"].T, ue[1]["b"], act="leaky")
    xs = _mm(x_server, se[0]["W"].T, se[0]["b"], act="relu")
    xs = _mm(xs, se[1]["W"].T, se[1]["b"], act="leaky")

    for cp in params["convs"]:
        u1 = _conv_rel(cp["s2u"], xs, xu, s2u_src, s2u_dst, edge_attr_s2u,
                       n_user, True)
        u2 = _conv_rel(cp["u2u"], xu, xu, u2u_src, u2u_dst, edge_attr_u2u,
                       n_user, False)
        xu = u1 + u2

    off = params["off"]
    o1 = _mm(xu, off[0]["W"].T, off[0]["b"], act="sigmoid")
    ol = _mm(o1, off[1]["W"].T, off[1]["b"])
    probs = jax.nn.softmax(ol, axis=-1)

    sel = params["sel"]
    w0 = sel[0]["W"]
    p_user = (_mm(xu, w0[:, :H].T, sel[0]["b"])
              + probs @ w0[:, H:H + 2].T)
    q_srv = _mm(xs, w0[:, H + 2:].T, jnp.zeros((H,), jnp.float32))

    pad = E_PAD - E
    ug = jnp.concatenate([u2s_u, jnp.zeros((pad,), jnp.int32)])
    sg = jnp.concatenate([u2s_s, jnp.zeros((pad,), jnp.int32)])
    us = jnp.concatenate([u2s_u, jnp.full((pad,), PAD_DST, jnp.int32)])
    h1 = _sc_pair_sigmoid(p_user, q_srv, ug, sg)
    h2 = _mm(h1, sel[1]["W"].T, sel[1]["b"], act="sigmoid")
    aw = _mm(h2, sel[2]["W"].T, sel[2]["b"])[:, 0]
    mx2 = _sc_combine(_sc_segmax(aw, us).reshape(NW, N_PAD), False)
    ex2, s2 = _sc_segexp(aw, us, mx2)
    s2c = _sc_combine(s2.reshape(NW, N_PAD), True)
    scheme = _sc_segdiv(ex2, us, s2c)[:E]
    return probs, scheme


# speedup vs baseline: 2.8601x; 1.1955x over previous
import functools

import jax
import jax.numpy as jnp
from jax import lax
from jax.experimental import pallas as pl
from jax.experimental.pallas import tpu as pltpu
from jax.experimental.pallas import tpu_sc as plsc

H = 256
ALPHA = 0.02

NC, NS, L = 2, 16, 16
NW = NC * NS
E = 160000
EW = 5120
E_PAD = EW * NW
C = 128
CL = 64
CA = 64
N_PAD = 10240
PAD_DST = N_PAD - 1
QH = 64


def _vperm(x, idx):
    return lax.gather(
        x, idx[:, None],
        lax.GatherDimensionNumbers(offset_dims=(), collapsed_slice_dims=(0,),
                                   start_index_map=(0,)),
        slice_sizes=(1,), mode=lax.GatherScatterMode.PROMISE_IN_BOUNDS)


def _sc_logits_body(a_hbm, b_hbm, dstg_hbm, srcg_hbm, dsts_hbm, w2_hbm,
                    l_hbm, mx_hbm,
                    dstg_v, srcg_v, dsts_v, w2_v,
                    arows0, brows0, arows1, brows1, lbuf, maxacc, sema, semb):
    wid = lax.axis_index("s") * NC + lax.axis_index("c")
    base = wid * EW
    pltpu.sync_copy(dstg_hbm.at[pl.ds(base, EW)], dstg_v)
    pltpu.sync_copy(srcg_hbm.at[pl.ds(base, EW)], srcg_v)
    pltpu.sync_copy(dsts_hbm.at[pl.ds(base, EW)], dsts_v)
    pltpu.sync_copy(w2_hbm, w2_v)
    w2c = [w2_v[pl.ds(16 * i, 16)] for i in range(16)]
    lane = lax.broadcasted_iota(jnp.int32, (L,), 0)
    rots = [(lane + r) & (L - 1) for r in (8, 4, 2, 1)]

    def neg_init(i, _):
        maxacc[pl.ds(i * L, L)] = jnp.full((L,), -1e30, jnp.float32)
        return 0
    lax.fori_loop(0, N_PAD // L, neg_init, 0)

    def issue(c, arows, brows):
        pltpu.async_copy(a_hbm.at[dstg_v.at[pl.ds(c * CL, CL)]], arows, sema)
        pltpu.async_copy(b_hbm.at[srcg_v.at[pl.ds(c * CL, CL)]], brows, semb)

    def drain(arows, brows):
        pltpu.make_async_copy(a_hbm.at[pl.ds(0, CL)], arows, sema).wait()
        pltpu.make_async_copy(b_hbm.at[pl.ds(0, CL)], brows, semb).wait()

    def compute(c, arows, brows):
        def edge_body(e, lvec):
            acc = jnp.zeros((L,), jnp.float32)
            for kc in range(16):
                va = arows[e, pl.ds(kc * 16, 16)]
                vb = brows[e, pl.ds(kc * 16, 16)]
                g = va + vb
                g = jnp.where(g >= 0.0, g, ALPHA * g)
                acc = acc + g * w2c[kc]
            for r in rots:
                acc = acc + _vperm(acc, r)
            lvec = lvec + jnp.where(lane == (e % L), acc, 0.0)

            @pl.when(e % L == L - 1)
            def _():
                lbuf[pl.ds(c * CL + (e // L) * L, L)] = lvec
            return jnp.where(e % L == L - 1, jnp.zeros((L,), jnp.float32),
                             lvec)
        lax.fori_loop(0, CL, edge_body, jnp.zeros((L,), jnp.float32))

    nch2 = EW // CL // 2
    issue(0, arows0, brows0)

    def chunk2_body(c2, _):
        a = 2 * c2
        issue(a + 1, arows1, brows1)
        drain(arows0, brows0)
        compute(a, arows0, brows0)

        @pl.when(c2 + 1 < nch2)
        def _():
            issue(a + 2, arows0, brows0)
        drain(arows1, brows1)
        compute(a + 1, arows1, brows1)
        return 0
    lax.fori_loop(0, nch2, chunk2_body, 0)

    def seg_max(gi, _):
        lvec = lbuf[pl.ds(gi * L, L)]
        dstv = dsts_v[pl.ds(gi * L, L)]
        k, v = dstv, lvec
        for r in range(1, L):
            ridx = (lane + r) & (L - 1)
            km = _vperm(k, ridx)
            vm = _vperm(v, ridx)
            v = jnp.where(km == k, jnp.maximum(v, vm), v)
        cur = plsc.load_gather(maxacc, [k])
        plsc.store_scatter(maxacc, [k], jnp.maximum(cur, v))
        return 0
    lax.fori_loop(0, EW // L, seg_max, 0)

    pltpu.sync_copy(lbuf, l_hbm.at[pl.ds(base, EW)])
    pltpu.sync_copy(maxacc, mx_hbm.at[pl.ds(wid * N_PAD, N_PAD)])


def _sc_logits(a_tab, b_tab, dstg, srcg, dsts, w2):
    return pl.kernel(
        _sc_logits_body,
        out_type=(jax.ShapeDtypeStruct((E_PAD,), jnp.float32),
                  jax.ShapeDtypeStruct((NW * N_PAD,), jnp.float32)),
        mesh=plsc.VectorSubcoreMesh(core_axis_name="c", subcore_axis_name="s"),
        compiler_params=pltpu.CompilerParams(needs_layout_passes=False),
        scratch_types=[
            pltpu.VMEM((EW,), jnp.int32),
            pltpu.VMEM((EW,), jnp.int32),
            pltpu.VMEM((EW,), jnp.int32),
            pltpu.VMEM((H,), jnp.float32),
            pltpu.VMEM((CL, H), jnp.float32),
            pltpu.VMEM((CL, H), jnp.float32),
            pltpu.VMEM((CL, H), jnp.float32),
            pltpu.VMEM((CL, H), jnp.float32),
            pltpu.VMEM((EW,), jnp.float32),
            pltpu.VMEM((N_PAD,), jnp.float32),
            pltpu.SemaphoreType.DMA,
            pltpu.SemaphoreType.DMA,
        ],
    )(a_tab, b_tab, dstg, srcg, dsts, w2)


def _mm_body(x_ref, w_ref, b_ref, o_ref, *, act):
    acc = jnp.dot(x_ref[...], w_ref[...], preferred_element_type=jnp.float32)
    acc = acc + b_ref[...]
    if act == "relu":
        acc = jnp.maximum(acc, 0.0)
    elif act == "leaky":
        acc = jnp.where(acc >= 0.0, acc, ALPHA * acc)
    elif act == "sigmoid":
        acc = jax.nn.sigmoid(acc)
    o_ref[...] = acc


@functools.partial(jax.jit, static_argnames=("act", "bn"))
def _mm(x, wt, b, act="none", bn=1024):
    n, k = x.shape
    d = wt.shape[1]
    bn = min(bn, n)
    grid = (pl.cdiv(n, bn),)
    return pl.pallas_call(
        functools.partial(_mm_body, act=act),
        grid=grid,
        in_specs=[
            pl.BlockSpec((bn, k), lambda i: (i, 0)),
            pl.BlockSpec((k, d), lambda i: (0, 0)),
            pl.BlockSpec((1, d), lambda i: (0, 0)),
        ],
        out_specs=pl.BlockSpec((bn, d), lambda i: (i, 0)),
        out_shape=jax.ShapeDtypeStruct((n, d), jnp.float32),
    )(x, wt, b.reshape(1, d))


def _seg_softmax(logits, seg, num_segments):
    m = jax.ops.segment_max(logits, seg, num_segments)
    m = jnp.where(jnp.isfinite(m), m, 0.0)
    ex = jnp.exp(logits - m[seg])
    s = jax.ops.segment_sum(ex, seg, num_segments)
    return ex / (s[seg] + 1e-16)


def _sc_combine_body(part_hbm, out_hbm, mx_v, slice_v, *, is_sum):
    wid = lax.axis_index("s") * NC + lax.axis_index("c")
    nblk = N_PAD // 128

    def do_blk(blk):
        pltpu.sync_copy(part_hbm.at[:, pl.ds(blk * 128, 128)], mx_v)

        def grp(i, _):
            m = mx_v[0, pl.ds(i * L, L)]
            for w in range(1, NW):
                if is_sum:
                    m = m + mx_v[w, pl.ds(i * L, L)]
                else:
                    m = jnp.maximum(m, mx_v[w, pl.ds(i * L, L)])
            slice_v[pl.ds(i * L, L)] = m
            return 0
        lax.fori_loop(0, 128 // L, grp, 0)
        pltpu.sync_copy(slice_v, out_hbm.at[pl.ds(blk * 128, 128)])

    def blk_loop(i, _):
        blk = wid + i * NW

        @pl.when(blk < nblk)
        def _():
            do_blk(blk)
        return 0
    lax.fori_loop(0, (nblk + NW - 1) // NW, blk_loop, 0)


def _sc_combine(part, is_sum):
    return pl.kernel(
        functools.partial(_sc_combine_body, is_sum=is_sum),
        out_type=jax.ShapeDtypeStruct((N_PAD,), jnp.float32),
        mesh=plsc.VectorSubcoreMesh(core_axis_name="c", subcore_axis_name="s"),
        compiler_params=pltpu.CompilerParams(needs_layout_passes=False),
        scratch_types=[
            pltpu.VMEM((NW, 128), jnp.float32),
            pltpu.VMEM((128,), jnp.float32),
        ],
    )(part)


def _sc_aggr_body(l_hbm, m_hbm, dsts_hbm, dstg_hbm, ea_hbm, m0_hbm, m1_hbm,
                  wm_hbm, zeros_hbm,
                  s_hbm, ag_hbm,
                  m_buf, s_acc, dstg_v, lbuf, m_rows0, m_rows1, idxw0, idxw1,
                  eac0, eac1, wm_v, acc_sp, sema):
    cid = lax.axis_index("c")
    sid = lax.axis_index("s")
    wid = sid * NC + cid
    base = wid * EW

    pltpu.sync_copy(wm_hbm, wm_v)
    pltpu.sync_copy(m_hbm, m_buf)
    pltpu.sync_copy(dstg_hbm.at[pl.ds(base, EW)], dstg_v)
    pltpu.sync_copy(l_hbm.at[pl.ds(base, EW)], lbuf)

    def z_init(i, _):
        s_acc[pl.ds(i * L, L)] = jnp.zeros((L,), jnp.float32)
        return 0
    lax.fori_loop(0, N_PAD // L, z_init, 0)

    for h in range(2):
        m_tab = m0_hbm if h == 0 else m1_hbm
        pltpu.sync_copy(zeros_hbm, acc_sp.at[pl.ds(sid * 640, 640)])
        plsc.subcore_barrier()

        def stage(c, idxw, eac, m_rows):
            pltpu.sync_copy(dsts_hbm.at[pl.ds(base + c * CA, CA)], idxw)
            pltpu.sync_copy(ea_hbm.at[pl.ds(base + c * CA, CA)], eac)
            pltpu.async_copy(
                m_tab.at[dstg_v.at[pl.ds(c * CA, CA)]], m_rows, sema)

        def drain(m_rows):
            pltpu.make_async_copy(m0_hbm.at[pl.ds(0, CA)], m_rows,
                                  sema).wait()

        def compute(c, idxw, eac, m_rows):
            def grp_body(g, _):
                dv = idxw[pl.ds(g * L, L)]
                lv = lbuf[pl.ds(c * CA + g * L, L)]
                mv = plsc.load_gather(m_buf, [dv])
                ex = jnp.exp(lv - mv)
                if h == 0:
                    plsc.addupdate_scatter(s_acc, [dv], ex)
                eag = eac[pl.ds(g * L, L)]
                for el in range(L):
                    e = g * L + el
                    sel = jnp.full((L,), el, jnp.int32)
                    eav = _vperm(eag, sel)
                    exv = _vperm(ex, sel)
                    for kc in range(128 // L):
                        mr = m_rows[e, pl.ds(kc * L, L)]
                        wmv = wm_v[pl.ds(h * 128 + kc * L, L)]
                        m_rows[e, pl.ds(kc * L, L)] = (
                            jnp.maximum(mr + eav * wmv, 0.0) * exv)
                return 0
            lax.fori_loop(0, CA // L, grp_body, 0)
            pltpu.sync_copy(m_rows, acc_sp.at[idxw], add=True)

        nch2 = EW // CA // 2
        stage(0, idxw0, eac0, m_rows0)

        def chunk2_body(c2, _):
            a = 2 * c2
            stage(a + 1, idxw1, eac1, m_rows1)
            drain(m_rows0)
            compute(a, idxw0, eac0, m_rows0)

            @pl.when(c2 + 1 < nch2)
            def _():
                stage(a + 2, idxw0, eac0, m_rows0)
            drain(m_rows1)
            compute(a + 1, idxw1, eac1, m_rows1)
            return 0
        lax.fori_loop(0, nch2, chunk2_body, 0)
        plsc.subcore_barrier()
        pltpu.sync_copy(
            acc_sp.at[pl.ds(sid * 640, 640)],
            ag_hbm.at[pl.ds((h * NC + cid) * N_PAD + sid * 640, 640)])
        plsc.subcore_barrier()
    pltpu.sync_copy(s_acc, s_hbm.at[pl.ds(wid * N_PAD, N_PAD)])


def _sc_aggr(l_pad, m_comb, dsts, dstg, ea_pad, m_tabs, wm, zeros):
    return pl.kernel(
        _sc_aggr_body,
        out_type=(jax.ShapeDtypeStruct((NW * N_PAD,), jnp.float32),
                  jax.ShapeDtypeStruct((2 * NC * N_PAD, 128), jnp.float32)),
        mesh=plsc.VectorSubcoreMesh(core_axis_name="c", subcore_axis_name="s"),
        compiler_params=pltpu.CompilerParams(needs_layout_passes=False),
        scratch_types=[
            pltpu.VMEM((N_PAD,), jnp.float32),
            pltpu.VMEM((N_PAD,), jnp.float32),
            pltpu.VMEM((EW,), jnp.int32),
            pltpu.VMEM((EW,), jnp.float32),
            pltpu.VMEM((CA, 128), jnp.float32),
            pltpu.VMEM((CA, 128), jnp.float32),
            pltpu.VMEM((CA,), jnp.int32),
            pltpu.VMEM((CA,), jnp.int32),
            pltpu.VMEM((CA,), jnp.float32),
            pltpu.VMEM((CA,), jnp.float32),
            pltpu.VMEM((H,), jnp.float32),
            pltpu.VMEM_SHARED((N_PAD, 128), jnp.float32),
            pltpu.SemaphoreType.DMA,
        ],
    )(l_pad, m_comb, dsts, dstg, ea_pad, *m_tabs, wm, zeros)


def _sc_pair_body(p_hbm, q_hbm, ug_hbm, sg_hbm,
                  h1_hbm,
                  idxu_v, idxs_v, prows, qrows, semu, sems):
    wid = lax.axis_index("s") * NC + lax.axis_index("c")
    base = wid * EW

    def chunk_body(c, _):
        pltpu.sync_copy(ug_hbm.at[pl.ds(base + c * C, C)], idxu_v)
        pltpu.sync_copy(sg_hbm.at[pl.ds(base + c * C, C)], idxs_v)
        cpu_ = pltpu.async_copy(p_hbm.at[idxu_v], prows, semu)
        cps_ = pltpu.async_copy(q_hbm.at[idxs_v], qrows, sems)
        cpu_.wait()
        cps_.wait()

        def edge_body(e, _):
            for kc in range(16):
                x = prows[e, pl.ds(kc * L, L)] + qrows[e, pl.ds(kc * L, L)]
                prows[e, pl.ds(kc * L, L)] = 1.0 / (1.0 + jnp.exp(-x))
            return 0
        lax.fori_loop(0, C, edge_body, 0)
        pltpu.sync_copy(prows, h1_hbm.at[pl.ds(base + c * C, C)])
        return 0
    lax.fori_loop(0, EW // C, chunk_body, 0)


def _sc_pair_sigmoid(p_tab, q_tab, ug, sg):
    return pl.kernel(
        _sc_pair_body,
        out_type=jax.ShapeDtypeStruct((E_PAD, H), jnp.float32),
        mesh=plsc.VectorSubcoreMesh(core_axis_name="c", subcore_axis_name="s"),
        compiler_params=pltpu.CompilerParams(needs_layout_passes=False),
        scratch_types=[
            pltpu.VMEM((C,), jnp.int32),
            pltpu.VMEM((C,), jnp.int32),
            pltpu.VMEM((C, H), jnp.float32),
            pltpu.VMEM((C, H), jnp.float32),
            pltpu.SemaphoreType.DMA,
            pltpu.SemaphoreType.DMA,
        ],
    )(p_tab, q_tab, ug, sg)


def _sc_segmax_body(aw_hbm, seg_hbm, mx_hbm, awbuf, segbuf, maxacc):
    wid = lax.axis_index("s") * NC + lax.axis_index("c")
    base = wid * EW
    lane = lax.broadcasted_iota(jnp.int32, (L,), 0)
    pltpu.sync_copy(aw_hbm.at[pl.ds(base, EW)], awbuf)
    pltpu.sync_copy(seg_hbm.at[pl.ds(base, EW)], segbuf)

    def neg_init(i, _):
        maxacc[pl.ds(i * L, L)] = jnp.full((L,), -1e30, jnp.float32)
        return 0
    lax.fori_loop(0, N_PAD // L, neg_init, 0)

    def seg_max(gi, _):
        v = awbuf[pl.ds(gi * L, L)]
        k = segbuf[pl.ds(gi * L, L)]
        for r in range(1, L):
            ridx = (lane + r) & (L - 1)
            km = _vperm(k, ridx)
            vm = _vperm(v, ridx)
            v = jnp.where(km == k, jnp.maximum(v, vm), v)
        cur = plsc.load_gather(maxacc, [k])
        plsc.store_scatter(maxacc, [k], jnp.maximum(cur, v))
        return 0
    lax.fori_loop(0, EW // L, seg_max, 0)
    pltpu.sync_copy(maxacc, mx_hbm.at[pl.ds(wid * N_PAD, N_PAD)])


def _sc_segmax(aw, seg):
    return pl.kernel(
        _sc_segmax_body,
        out_type=jax.ShapeDtypeStruct((NW * N_PAD,), jnp.float32),
        mesh=plsc.VectorSubcoreMesh(core_axis_name="c", subcore_axis_name="s"),
        compiler_params=pltpu.CompilerParams(needs_layout_passes=False),
        scratch_types=[
            pltpu.VMEM((EW,), jnp.float32),
            pltpu.VMEM((EW,), jnp.int32),
            pltpu.VMEM((N_PAD,), jnp.float32),
        ],
    )(aw, seg)


def _sc_segexp_body(aw_hbm, seg_hbm, m_hbm, ex_hbm, s_hbm,
                    awbuf, segbuf, m_buf, s_acc):
    wid = lax.axis_index("s") * NC + lax.axis_index("c")
    base = wid * EW
    pltpu.sync_copy(aw_hbm.at[pl.ds(base, EW)], awbuf)
    pltpu.sync_copy(seg_hbm.at[pl.ds(base, EW)], segbuf)
    pltpu.sync_copy(m_hbm, m_buf)

    def z_init(i, _):
        s_acc[pl.ds(i * L, L)] = jnp.zeros((L,), jnp.float32)
        return 0
    lax.fori_loop(0, N_PAD // L, z_init, 0)

    def grp(gi, _):
        k = segbuf[pl.ds(gi * L, L)]
        mv = plsc.load_gather(m_buf, [k])
        ex = jnp.exp(awbuf[pl.ds(gi * L, L)] - mv)
        plsc.addupdate_scatter(s_acc, [k], ex)
        awbuf[pl.ds(gi * L, L)] = ex
        return 0
    lax.fori_loop(0, EW // L, grp, 0)
    pltpu.sync_copy(awbuf, ex_hbm.at[pl.ds(base, EW)])
    pltpu.sync_copy(s_acc, s_hbm.at[pl.ds(wid * N_PAD, N_PAD)])


def _sc_segexp(aw, seg, m_comb):
    return pl.kernel(
        _sc_segexp_body,
        out_type=(jax.ShapeDtypeStruct((E_PAD,), jnp.float32),
                  jax.ShapeDtypeStruct((NW * N_PAD,), jnp.float32)),
        mesh=plsc.VectorSubcoreMesh(core_axis_name="c", subcore_axis_name="s"),
        compiler_params=pltpu.CompilerParams(needs_layout_passes=False),
        scratch_types=[
            pltpu.VMEM((EW,), jnp.float32),
            pltpu.VMEM((EW,), jnp.int32),
            pltpu.VMEM((N_PAD,), jnp.float32),
            pltpu.VMEM((N_PAD,), jnp.float32),
        ],
    )(aw, seg, m_comb)


def _sc_segdiv_body(ex_hbm, seg_hbm, s_hbm, out_hbm,
                    exbuf, segbuf, s_buf):
    wid = lax.axis_index("s") * NC + lax.axis_index("c")
    base = wid * EW
    pltpu.sync_copy(ex_hbm.at[pl.ds(base, EW)], exbuf)
    pltpu.sync_copy(seg_hbm.at[pl.ds(base, EW)], segbuf)
    pltpu.sync_copy(s_hbm, s_buf)

    def grp(gi, _):
        k = segbuf[pl.ds(gi * L, L)]
        sv = plsc.load_gather(s_buf, [k])
        exbuf[pl.ds(gi * L, L)] = exbuf[pl.ds(gi * L, L)] / (sv + 1e-16)
        return 0
    lax.fori_loop(0, EW // L, grp, 0)
    pltpu.sync_copy(exbuf, out_hbm.at[pl.ds(base, EW)])


def _sc_segdiv(ex, seg, s_comb):
    return pl.kernel(
        _sc_segdiv_body,
        out_type=jax.ShapeDtypeStruct((E_PAD,), jnp.float32),
        mesh=plsc.VectorSubcoreMesh(core_axis_name="c", subcore_axis_name="s"),
        compiler_params=pltpu.CompilerParams(needs_layout_passes=False),
        scratch_types=[
            pltpu.VMEM((EW,), jnp.float32),
            pltpu.VMEM((EW,), jnp.int32),
            pltpu.VMEM((N_PAD,), jnp.float32),
        ],
    )(ex, seg, s_comb)


def _pad_edges(src, dst):
    pad = E_PAD - src.shape[0]
    src_g = jnp.concatenate([src, jnp.zeros((pad,), jnp.int32)])
    dst_g = jnp.concatenate([dst, jnp.zeros((pad,), jnp.int32)])
    dst_s = jnp.concatenate([dst, jnp.full((pad,), PAD_DST, jnp.int32)])
    return src_g, dst_g, dst_s


def _conv_rel(p, x_src, x_dst, src, dst, ea, n_dst, update_relu):
    w_att1 = p["att1"]["W"]
    w1a, w1b = w_att1[:, :H], w_att1[:, H:]
    b1 = p["att1"]["b"]
    wq, bq = p["Wq"]["W"], p["Wq"]["b"]
    wr, br = p["Wr"]["W"], p["Wr"]["b"]
    wa = jnp.dot(w1a, wq).T
    wb = jnp.dot(w1b, wr).T
    ba = jnp.dot(w1a, bq) + b1
    bb = jnp.dot(w1b, br)
    a_node = _mm(x_dst, wa, ba)
    b_node = _mm(x_src, wb, bb)

    w_msg = p["msg"]["W"]
    m_node = _mm(x_dst, w_msg[:, :H].T, p["msg"]["b"])
    wm_e = w_msg[:, H]

    w2 = p["att2"]["W"][0]

    src_g, dst_g, dst_s = _pad_edges(src, dst)
    l_pad, maxes = _sc_logits(a_node, b_node, dst_g, src_g, dst_s, w2)

    ea_pad = jnp.concatenate(
        [ea[:, 0], jnp.zeros((E_PAD - E,), jnp.float32)])
    m_tabs = [m_node[:, :128], m_node[:, 128:]]
    zeros = jnp.zeros((640, 128), jnp.float32)
    m_comb = _sc_combine(maxes.reshape(NW, N_PAD), False)
    s_part, ag_part = _sc_aggr(l_pad, m_comb, dst_s, dst_g,
                               ea_pad, m_tabs, wm_e, zeros)
    s = s_part.reshape(NW, N_PAD).sum(axis=0)[:n_dst]
    ap = ag_part.reshape(2, NC, N_PAD, 128)
    aggr_raw = jnp.concatenate([ap[0, 0] + ap[0, 1], ap[1, 0] + ap[1, 1]],
                               axis=1)[:n_dst]
    aggr = aggr_raw / (s + 1e-16)[:, None]

    w_upd = p["upd"]["W"]
    upd = _mm(jnp.concatenate([aggr, x_dst], axis=1), w_upd.T, p["upd"]["b"])
    return jnp.maximum(upd, 0.0) if update_relu else upd


def kernel(x_user, x_server, edge_index_s2u, edge_index_u2u, edge_index_u2s,
           edge_attr_s2u, edge_attr_u2u, params):
    n_user = x_user.shape[0]
    s2u_src = edge_index_s2u[0].astype(jnp.int32)
    s2u_dst = edge_index_s2u[1].astype(jnp.int32)
    u2u_src = edge_index_u2u[0].astype(jnp.int32)
    u2u_dst = edge_index_u2u[1].astype(jnp.int32)
    u2s_u = edge_index_u2s[0].astype(jnp.int32)
    u2s_s = edge_index_u2s[1].astype(jnp.int32)

    ue, se = params["user_enc"], params["server_enc"]
    xu = _mm(x_user, ue[0]["W"].T, ue[0]["b"], act="relu")
    xu = _mm(xu, ue[1]["W"].T, ue[1]["b"], act="leaky")
    xs = _mm(x_server, se[0]["W"].T, se[0]["b"], act="relu")
    xs = _mm(xs, se[1]["W"].T, se[1]["b"], act="leaky")

    for cp in params["convs"]:
        u1 = _conv_rel(cp["s2u"], xs, xu, s2u_src, s2u_dst, edge_attr_s2u,
                       n_user, True)
        u2 = _conv_rel(cp["u2u"], xu, xu, u2u_src, u2u_dst, edge_attr_u2u,
                       n_user, False)
        xu = u1 + u2

    off = params["off"]
    o1 = _mm(xu, off[0]["W"].T, off[0]["b"], act="sigmoid")
    ol = _mm(o1, off[1]["W"].T, off[1]["b"])
    probs = jax.nn.softmax(ol, axis=-1)

    sel = params["sel"]
    w0 = sel[0]["W"]
    p_user = (_mm(xu, w0[:, :H].T, sel[0]["b"])
              + probs @ w0[:, H:H + 2].T)
    q_srv = _mm(xs, w0[:, H + 2:].T, jnp.zeros((H,), jnp.float32))

    pad = E_PAD - E
    ug = jnp.concatenate([u2s_u, jnp.zeros((pad,), jnp.int32)])
    sg = jnp.concatenate([u2s_s, jnp.zeros((pad,), jnp.int32)])
    us = jnp.concatenate([u2s_u, jnp.full((pad,), PAD_DST, jnp.int32)])
    h1 = _sc_pair_sigmoid(p_user, q_srv, ug, sg)
    h2 = _mm(h1, sel[1]["W"].T, sel[1]["b"], act="sigmoid")
    aw = _mm(h2, sel[2]["W"].T, sel[2]["b"])[:, 0]
    mx2 = _sc_combine(_sc_segmax(aw, us).reshape(NW, N_PAD), False)
    ex2, s2 = _sc_segexp(aw, us, mx2)
    s2c = _sc_combine(s2.reshape(NW, N_PAD), True)
    scheme = _sc_segdiv(ex2, us, s2c)[:E]
    return probs, scheme
